# Initial kernel scaffold; baseline (speedup 1.0000x reference)
import jax
import jax.numpy as jnp
from jax.experimental import pallas as pl


def kernel(xyz, points, conv1_params, conv2_params):
    raise NotImplementedError("write your pallas kernel here")


# speedup vs baseline: 4.1020x; 4.1020x over previous
import functools

import jax
import jax.numpy as jnp
from jax import lax
from jax.experimental import pallas as pl
from jax.experimental.pallas import tpu as pltpu
from jax.experimental.pallas import tpu_sc as plsc

_RADIUS = 0.2
_R2 = _RADIUS * _RADIUS
_EPS = 1e-5


def _select_body(xyz8_ref, xyzt8_ref, out_ref, *, n_total, bn_total, cb):
    b = pl.program_id(0)
    nb = pl.program_id(1)
    ctr = xyz8_ref[0]
    xt = xyzt8_ref[0]
    colid = lax.broadcasted_iota(jnp.int32, (cb, n_total), 1)
    rowid = lax.broadcasted_iota(jnp.int32, (cb, n_total), 0) + nb * cb
    d0 = xt[0:1, :] - ctr[:, 0:1]
    d1 = xt[1:2, :] - ctr[:, 1:2]
    d2 = xt[2:3, :] - ctr[:, 2:3]
    dist = (d0 * d0 + d1 * d1) + d2 * d2
    r2 = jnp.float32(_R2)
    big = jnp.float32(1e10)
    base = (dist > 1e-10) & (dist < r2)
    eye = colid == rowid
    s0 = d0 >= 0.0
    s1 = d1 >= 0.0
    s2 = d2 >= 0.0
    for i in range(8):
        octm = ((s0 if (i & 4) else jnp.logical_not(s0))
                & (s1 if (i & 2) else jnp.logical_not(s1))
                & (s2 if (i & 1) else jnp.logical_not(s2)))
        val = jnp.where(base & octm, dist, big)
        val = jnp.where(eye, r2, val)
        mn = jnp.min(val, axis=1, keepdims=True)
        cand = jnp.where(val == mn, colid, jnp.int32(n_total))
        pj = jnp.min(cand, axis=1)
        out_ref[0, i, :] = pj + (b * n_total + (i & 1) * bn_total)


def _select_call(xyz8, xyzt8, cb=256):
    b, n, _ = xyz8.shape
    return pl.pallas_call(
        functools.partial(_select_body, n_total=n, bn_total=b * n, cb=cb),
        grid=(b, n // cb),
        in_specs=[
            pl.BlockSpec((1, cb, 8), lambda bi, ni: (bi, ni, 0)),
            pl.BlockSpec((1, 8, n), lambda bi, ni: (bi, 0, 0)),
        ],
        out_specs=pl.BlockSpec((1, 8, cb), lambda bi, ni: (bi, 0, ni)),
        out_shape=jax.ShapeDtypeStruct((b, 8, n), jnp.int32),
    )(xyz8, xyzt8)


def _table_body(feat_ref, xyz8_ref, wf_ref, wx_ref, out_ref):
    out_ref[0] = (jnp.dot(feat_ref[...], wf_ref[0],
                          preferred_element_type=jnp.float32)
                  + jnp.dot(xyz8_ref[...], wx_ref[0],
                            preferred_element_type=jnp.float32))


def _table_affine_body(y_ref, s_ref, t_ref, xyz8_ref, wf_ref, wx_ref, out_ref):
    z = jnp.maximum(y_ref[...] * s_ref[...] + t_ref[...], 0.0)
    out_ref[0] = (jnp.dot(z, wf_ref[0], preferred_element_type=jnp.float32)
                  + jnp.dot(xyz8_ref[...], wx_ref[0],
                            preferred_element_type=jnp.float32))


def _table_call(feat, xyz8f, wf, wx, rb=512):
    m, c = feat.shape
    o = wf.shape[-1]
    return pl.pallas_call(
        _table_body,
        grid=(2, m // rb),
        in_specs=[
            pl.BlockSpec((rb, c), lambda p, i: (i, 0)),
            pl.BlockSpec((rb, 8), lambda p, i: (i, 0)),
            pl.BlockSpec((1, c, o), lambda p, i: (p, 0, 0)),
            pl.BlockSpec((1, 8, o), lambda p, i: (p, 0, 0)),
        ],
        out_specs=pl.BlockSpec((1, rb, o), lambda p, i: (p, i, 0)),
        out_shape=jax.ShapeDtypeStruct((2, m, o), jnp.float32),
    )(feat, xyz8f, wf, wx)


def _table_affine_call(y, s, t, xyz8f, wf, wx, rb=512):
    m, c = y.shape
    o = wf.shape[-1]
    return pl.pallas_call(
        _table_affine_body,
        grid=(2, m // rb),
        in_specs=[
            pl.BlockSpec((rb, c), lambda p, i: (i, 0)),
            pl.BlockSpec((1, c), lambda p, i: (0, 0)),
            pl.BlockSpec((1, c), lambda p, i: (0, 0)),
            pl.BlockSpec((rb, 8), lambda p, i: (i, 0)),
            pl.BlockSpec((1, c, o), lambda p, i: (p, 0, 0)),
            pl.BlockSpec((1, 8, o), lambda p, i: (p, 0, 0)),
        ],
        out_specs=pl.BlockSpec((1, rb, o), lambda p, i: (p, i, 0)),
        out_shape=jax.ShapeDtypeStruct((2, m, o), jnp.float32),
    )(y, s, t, xyz8f, wf, wx)


def _sc_gather_call(table, idx_flat):
    v, d = table.shape
    (r,) = idx_flat.shape
    nc, ns = 2, 16
    nw = nc * ns
    per_w = r // nw
    ch = 128
    nchunk = per_w // ch
    mesh = plsc.VectorSubcoreMesh(core_axis_name="c", subcore_axis_name="s")

    @functools.partial(
        pl.kernel,
        out_type=jax.ShapeDtypeStruct((r, d), jnp.float32),
        mesh=mesh,
        scratch_types=[
            pltpu.VMEM((ch,), jnp.int32),
            pltpu.VMEM((ch, d), jnp.float32),
            pltpu.SemaphoreType.DMA,
        ],
    )
    def gk(table_hbm, idx_hbm, out_hbm, idx_v, row_v, sem):
        wid = lax.axis_index("s") * nc + lax.axis_index("c")
        base = wid * per_w
        for i in range(nchunk):
            off = base + i * ch
            pltpu.sync_copy(idx_hbm.at[pl.ds(off, ch)], idx_v)
            pltpu.async_copy(table_hbm.at[idx_v], row_v, sem).wait()
            pltpu.sync_copy(row_v, out_hbm.at[pl.ds(off, ch)])

    return gk(table, idx_flat)


def _pairsum_body(g_ref, xr_ref, wxn_ref, bias_ref, gam_ref, bet_ref,
                  y_ref, s_ref, t_ref, acc, *, nsteps, count, d):
    i = pl.program_id(0)

    @pl.when(i == 0)
    def _init():
        acc[...] = jnp.zeros_like(acc)

    g = g_ref[...]
    y = (g[:, :d] + g[:, d:]
         + jnp.dot(xr_ref[...], wxn_ref[...], preferred_element_type=jnp.float32)
         + bias_ref[...])
    y_ref[...] = y
    acc[0:1, :] = acc[0:1, :] + jnp.sum(y, axis=0, keepdims=True)
    acc[1:2, :] = acc[1:2, :] + jnp.sum(y * y, axis=0, keepdims=True)

    @pl.when(i == nsteps - 1)
    def _fin():
        inv = jnp.float32(1.0 / count)
        mean = acc[0:1, :] * inv
        var = acc[1:2, :] * inv - mean * mean
        sc = gam_ref[...] / jnp.sqrt(var + _EPS)
        s_ref[...] = sc
        t_ref[...] = bet_ref[...] - mean * sc


def _pairsum_call(gp, xyzrep, wxn, bias, gam, bet, rb=1024):
    m, twod = gp.shape
    d = twod // 2
    nsteps = m // rb
    return pl.pallas_call(
        functools.partial(_pairsum_body, nsteps=nsteps, count=m, d=d),
        grid=(nsteps,),
        in_specs=[
            pl.BlockSpec((rb, twod), lambda i: (i, 0)),
            pl.BlockSpec((rb, 8), lambda i: (i, 0)),
            pl.BlockSpec((8, d), lambda i: (0, 0)),
            pl.BlockSpec((1, d), lambda i: (0, 0)),
            pl.BlockSpec((1, d), lambda i: (0, 0)),
            pl.BlockSpec((1, d), lambda i: (0, 0)),
        ],
        out_specs=[
            pl.BlockSpec((rb, d), lambda i: (i, 0)),
            pl.BlockSpec((1, d), lambda i: (0, 0)),
            pl.BlockSpec((1, d), lambda i: (0, 0)),
        ],
        out_shape=[
            jax.ShapeDtypeStruct((m, d), jnp.float32),
            jax.ShapeDtypeStruct((1, d), jnp.float32),
            jax.ShapeDtypeStruct((1, d), jnp.float32),
        ],
        scratch_shapes=[pltpu.VMEM((2, d), jnp.float32)],
    )(gp, xyzrep, wxn, bias, gam, bet)


def _pairmm_body(y_ref, sp_ref, tp_ref, w_ref, b_ref, gam_ref, bet_ref,
                 o_ref, s_ref, t_ref, acc, *, nsteps, count):
    i = pl.program_id(0)

    @pl.when(i == 0)
    def _init():
        acc[...] = jnp.zeros_like(acc)

    sp = sp_ref[...]
    tp = tp_ref[...]
    s2 = jnp.concatenate([sp, sp], axis=1)
    t2 = jnp.concatenate([tp, tp], axis=1)
    z = jnp.maximum(y_ref[...] * s2 + t2, 0.0)
    y = jnp.dot(z, w_ref[...], preferred_element_type=jnp.float32) + b_ref[...]
    o_ref[...] = y
    acc[0:1, :] = acc[0:1, :] + jnp.sum(y, axis=0, keepdims=True)
    acc[1:2, :] = acc[1:2, :] + jnp.sum(y * y, axis=0, keepdims=True)

    @pl.when(i == nsteps - 1)
    def _fin():
        inv = jnp.float32(1.0 / count)
        mean = acc[0:1, :] * inv
        var = acc[1:2, :] * inv - mean * mean
        sc = gam_ref[...] / jnp.sqrt(var + _EPS)
        s_ref[...] = sc
        t_ref[...] = bet_ref[...] - mean * sc


def _pairmm_call(yp, sp, tp, w, bias, gam, bet, rb=1024):
    m, twod = yp.shape
    d = twod // 2
    nsteps = m // rb
    return pl.pallas_call(
        functools.partial(_pairmm_body, nsteps=nsteps, count=m),
        grid=(nsteps,),
        in_specs=[
            pl.BlockSpec((rb, twod), lambda i: (i, 0)),
            pl.BlockSpec((1, d), lambda i: (0, 0)),
            pl.BlockSpec((1, d), lambda i: (0, 0)),
            pl.BlockSpec((twod, d), lambda i: (0, 0)),
            pl.BlockSpec((1, d), lambda i: (0, 0)),
            pl.BlockSpec((1, d), lambda i: (0, 0)),
            pl.BlockSpec((1, d), lambda i: (0, 0)),
        ],
        out_specs=[
            pl.BlockSpec((rb, d), lambda i: (i, 0)),
            pl.BlockSpec((1, d), lambda i: (0, 0)),
            pl.BlockSpec((1, d), lambda i: (0, 0)),
        ],
        out_shape=[
            jax.ShapeDtypeStruct((m, d), jnp.float32),
            jax.ShapeDtypeStruct((1, d), jnp.float32),
            jax.ShapeDtypeStruct((1, d), jnp.float32),
        ],
        scratch_shapes=[pltpu.VMEM((2, d), jnp.float32)],
    )(yp, sp, tp, w, bias, gam, bet)


def _final_body(y_ref, s_ref, t_ref, p_ref, o_ref):
    o_ref[...] = jnp.maximum(
        y_ref[...] * s_ref[...] + t_ref[...] + p_ref[...], 0.0)


def _final_call(y, s, t, pts, rb=1024):
    m, d = y.shape
    return pl.pallas_call(
        _final_body,
        grid=(m // rb,),
        in_specs=[
            pl.BlockSpec((rb, d), lambda i: (i, 0)),
            pl.BlockSpec((1, d), lambda i: (0, 0)),
            pl.BlockSpec((1, d), lambda i: (0, 0)),
            pl.BlockSpec((rb, d), lambda i: (i, 0)),
        ],
        out_specs=pl.BlockSpec((rb, d), lambda i: (i, 0)),
        out_shape=jax.ShapeDtypeStruct((m, d), jnp.float32),
    )(y, s, t, pts)


def _tap_weights(w):
    wx = jnp.stack([w[:, :3, 0, 0].T, w[:, :3, 0, 1].T])
    wx8 = jnp.pad(wx, ((0, 0), (0, 5), (0, 0)))
    wf = jnp.stack([w[:, 3:, 0, 0].T, w[:, 3:, 0, 1].T])
    return wx8, wf


def _pair_weight(w):
    return jnp.concatenate([w[:, :, 0, 0].T, w[:, :, 0, 1].T], axis=0)


def kernel(xyz, points, conv1_params, conv2_params):
    b, n, _ = xyz.shape
    c = points.shape[-1]
    m = b * n

    xyz8 = jnp.pad(xyz, ((0, 0), (0, 0), (0, 5)))
    xyzt8 = jnp.transpose(xyz8, (0, 2, 1))
    idxg = _select_call(xyz8, xyzt8)
    idx_flat = jnp.transpose(idxg, (0, 2, 1)).reshape(-1)

    xyz8f = xyz8.reshape(m, 8)
    pts = points.reshape(m, c)
    xyzrep = jnp.broadcast_to(
        xyz8.reshape(b, n, 1, 8), (b, n, 4, 8)).reshape(4 * m, 8)

    (w1, b1, g1, be1), (w2, b2, g2, be2), (w3, b3, g3, be3) = conv1_params
    (v1, c1, h1, ce1), (v2, c2, h2, ce2), (v3, c3, h3, ce3) = conv2_params

    wx8, wf = _tap_weights(w1)
    tt = _table_call(pts, xyz8f, wf, wx8)
    o = tt.shape[-1]
    gg = _sc_gather_call(tt.reshape(2 * m, o), idx_flat)
    wxn = -(wx8[0] + wx8[1])
    y1, s1, t1 = _pairsum_call(gg.reshape(4 * m, 2 * o), xyzrep, wxn,
                               b1[None], g1[None], be1[None])
    y2, s2, t2 = _pairmm_call(y1.reshape(2 * m, 2 * o), s1, t1,
                              _pair_weight(w2), b2[None], g2[None], be2[None])
    y3, s3, t3 = _pairmm_call(y2.reshape(m, 2 * o), s2, t2,
                              _pair_weight(w3), b3[None], g3[None], be3[None])

    vx8, vf = _tap_weights(v1)
    uu = _table_affine_call(y3, s3, t3, xyz8f, vf, vx8)
    hh = _sc_gather_call(uu.reshape(2 * m, o), idx_flat)
    vxn = -(vx8[0] + vx8[1])
    y4, s4, t4 = _pairsum_call(hh.reshape(4 * m, 2 * o), xyzrep, vxn,
                               c1[None], h1[None], ce1[None])
    y5, s5, t5 = _pairmm_call(y4.reshape(2 * m, 2 * o), s4, t4,
                              _pair_weight(v2), c2[None], h2[None], ce2[None])
    y6, s6, t6 = _pairmm_call(y5.reshape(m, 2 * o), s5, t5,
                              _pair_weight(v3), c3[None], h3[None], ce3[None])

    out = _final_call(y6, s6, t6, pts)
    return (xyz, out.reshape(b, n, o))


# speedup vs baseline: 4.5386x; 1.1064x over previous
import functools

import jax
import jax.numpy as jnp
from jax import lax
from jax.experimental import pallas as pl
from jax.experimental.pallas import tpu as pltpu
from jax.experimental.pallas import tpu_sc as plsc

_RADIUS = 0.2
_R2 = _RADIUS * _RADIUS
_EPS = 1e-5


def _select_body(xyz8_ref, xyzt8_ref, out_ref, *, n_total, bn_total, cb):
    b = pl.program_id(0)
    nb = pl.program_id(1)
    ctr = xyz8_ref[0]
    xt = xyzt8_ref[0]
    colid = lax.broadcasted_iota(jnp.int32, (cb, n_total), 1)
    rowid = lax.broadcasted_iota(jnp.int32, (cb, n_total), 0) + nb * cb
    d0 = xt[0:1, :] - ctr[:, 0:1]
    d1 = xt[1:2, :] - ctr[:, 1:2]
    d2 = xt[2:3, :] - ctr[:, 2:3]
    dist = (d0 * d0 + d1 * d1) + d2 * d2
    r2 = jnp.float32(_R2)
    big = jnp.float32(1e10)
    base = (dist > 1e-10) & (dist < r2)
    eye = colid == rowid
    sub = ((jnp.where(d0 >= 0.0, jnp.int32(4), jnp.int32(0))
            + jnp.where(d1 >= 0.0, jnp.int32(2), jnp.int32(0)))
           + jnp.where(d2 >= 0.0, jnp.int32(1), jnp.int32(0)))
    val_base = jnp.where(base, dist, big)
    val_base = jnp.where(eye, r2, val_base)
    for i in range(8):
        sel = (sub == i) | eye
        val = jnp.where(sel, val_base, big)
        mn = jnp.min(val, axis=1, keepdims=True)
        cand = jnp.where(val == mn, colid, jnp.int32(n_total))
        pj = jnp.min(cand, axis=1)
        out_ref[0, i, :] = pj + (b * n_total + (i & 1) * bn_total)


def _select_call(xyz8, xyzt8, cb=256):
    b, n, _ = xyz8.shape
    return pl.pallas_call(
        functools.partial(_select_body, n_total=n, bn_total=b * n, cb=cb),
        grid=(b, n // cb),
        in_specs=[
            pl.BlockSpec((1, cb, 8), lambda bi, ni: (bi, ni, 0)),
            pl.BlockSpec((1, 8, n), lambda bi, ni: (bi, 0, 0)),
        ],
        out_specs=pl.BlockSpec((1, 8, cb), lambda bi, ni: (bi, 0, ni)),
        out_shape=jax.ShapeDtypeStruct((b, 8, n), jnp.int32),
    )(xyz8, xyzt8)


def _table_body(feat_ref, xyz8_ref, wf_ref, wx_ref, out_ref):
    out_ref[0] = (jnp.dot(feat_ref[...], wf_ref[0],
                          preferred_element_type=jnp.float32)
                  + jnp.dot(xyz8_ref[...], wx_ref[0],
                            preferred_element_type=jnp.float32))


def _table_affine_body(y_ref, s_ref, t_ref, xyz8_ref, wf_ref, wx_ref, out_ref):
    z = jnp.maximum(y_ref[...] * s_ref[...] + t_ref[...], 0.0)
    out_ref[0] = (jnp.dot(z, wf_ref[0], preferred_element_type=jnp.float32)
                  + jnp.dot(xyz8_ref[...], wx_ref[0],
                            preferred_element_type=jnp.float32))


def _table_call(feat, xyz8f, wf, wx, rb=512):
    m, c = feat.shape
    o = wf.shape[-1]
    return pl.pallas_call(
        _table_body,
        grid=(2, m // rb),
        in_specs=[
            pl.BlockSpec((rb, c), lambda p, i: (i, 0)),
            pl.BlockSpec((rb, 8), lambda p, i: (i, 0)),
            pl.BlockSpec((1, c, o), lambda p, i: (p, 0, 0)),
            pl.BlockSpec((1, 8, o), lambda p, i: (p, 0, 0)),
        ],
        out_specs=pl.BlockSpec((1, rb, o), lambda p, i: (p, i, 0)),
        out_shape=jax.ShapeDtypeStruct((2, m, o), jnp.float32),
    )(feat, xyz8f, wf, wx)


def _table_affine_call(y, s, t, xyz8f, wf, wx, rb=512):
    m, c = y.shape
    o = wf.shape[-1]
    return pl.pallas_call(
        _table_affine_body,
        grid=(2, m // rb),
        in_specs=[
            pl.BlockSpec((rb, c), lambda p, i: (i, 0)),
            pl.BlockSpec((1, c), lambda p, i: (0, 0)),
            pl.BlockSpec((1, c), lambda p, i: (0, 0)),
            pl.BlockSpec((rb, 8), lambda p, i: (i, 0)),
            pl.BlockSpec((1, c, o), lambda p, i: (p, 0, 0)),
            pl.BlockSpec((1, 8, o), lambda p, i: (p, 0, 0)),
        ],
        out_specs=pl.BlockSpec((1, rb, o), lambda p, i: (p, i, 0)),
        out_shape=jax.ShapeDtypeStruct((2, m, o), jnp.float32),
    )(y, s, t, xyz8f, wf, wx)


def _sc_gather_call(table, idx_flat):
    v, d = table.shape
    (r,) = idx_flat.shape
    nc, ns = 2, 16
    nw = nc * ns
    per_w = r // nw
    ch = 128
    nchunk = per_w // ch
    mesh = plsc.VectorSubcoreMesh(core_axis_name="c", subcore_axis_name="s")

    @functools.partial(
        pl.kernel,
        out_type=jax.ShapeDtypeStruct((r, d), jnp.float32),
        mesh=mesh,
        scratch_types=[
            pltpu.VMEM((ch,), jnp.int32),
            pltpu.VMEM((ch,), jnp.int32),
            pltpu.VMEM((ch, d), jnp.float32),
            pltpu.VMEM((ch, d), jnp.float32),
            pltpu.SemaphoreType.DMA,
            pltpu.SemaphoreType.DMA,
            pltpu.SemaphoreType.DMA,
            pltpu.SemaphoreType.DMA,
        ],
    )
    def gk(table_hbm, idx_hbm, out_hbm, idx0, idx1, row0, row1,
           g0, g1, w0, w1):
        wid = lax.axis_index("s") * nc + lax.axis_index("c")
        base = wid * per_w
        idx_v = (idx0, idx1)
        row_v = (row0, row1)
        gsem = (g0, g1)
        wsem = (w0, w1)
        gh = [None] * nchunk
        wh = [None] * nchunk
        pltpu.sync_copy(idx_hbm.at[pl.ds(base, ch)], idx0)
        gh[0] = pltpu.async_copy(table_hbm.at[idx0], row0, g0)
        for i in range(nchunk):
            cur = i % 2
            nxt = 1 - cur
            if i + 1 < nchunk:
                pltpu.sync_copy(
                    idx_hbm.at[pl.ds(base + (i + 1) * ch, ch)], idx_v[nxt])
            gh[i].wait()
            if i > 0:
                wh[i - 1].wait()
            wh[i] = pltpu.async_copy(
                row_v[cur], out_hbm.at[pl.ds(base + i * ch, ch)], wsem[cur])
            if i + 1 < nchunk:
                gh[i + 1] = pltpu.async_copy(
                    table_hbm.at[idx_v[nxt]], row_v[nxt], gsem[nxt])
        wh[nchunk - 1].wait()

    return gk(table, idx_flat)


def _pairsum_body(g_ref, xr_ref, wxn_ref, bias_ref, gam_ref, bet_ref,
                  y_ref, s_ref, t_ref, acc, *, nsteps, count, d):
    i = pl.program_id(0)

    @pl.when(i == 0)
    def _init():
        acc[...] = jnp.zeros_like(acc)

    g = g_ref[...]
    y = (g[:, :d] + g[:, d:]
         + jnp.dot(xr_ref[...], wxn_ref[...], preferred_element_type=jnp.float32)
         + bias_ref[...])
    y_ref[...] = y
    acc[0:1, :] = acc[0:1, :] + jnp.sum(y, axis=0, keepdims=True)
    acc[1:2, :] = acc[1:2, :] + jnp.sum(y * y, axis=0, keepdims=True)

    @pl.when(i == nsteps - 1)
    def _fin():
        inv = jnp.float32(1.0 / count)
        mean = acc[0:1, :] * inv
        var = acc[1:2, :] * inv - mean * mean
        sc = gam_ref[...] / jnp.sqrt(var + _EPS)
        s_ref[...] = sc
        t_ref[...] = bet_ref[...] - mean * sc


def _pairsum_call(gp, xyzrep, wxn, bias, gam, bet, rb=1024):
    m, twod = gp.shape
    d = twod // 2
    nsteps = m // rb
    return pl.pallas_call(
        functools.partial(_pairsum_body, nsteps=nsteps, count=m, d=d),
        grid=(nsteps,),
        in_specs=[
            pl.BlockSpec((rb, twod), lambda i: (i, 0)),
            pl.BlockSpec((rb, 8), lambda i: (i, 0)),
            pl.BlockSpec((8, d), lambda i: (0, 0)),
            pl.BlockSpec((1, d), lambda i: (0, 0)),
            pl.BlockSpec((1, d), lambda i: (0, 0)),
            pl.BlockSpec((1, d), lambda i: (0, 0)),
        ],
        out_specs=[
            pl.BlockSpec((rb, d), lambda i: (i, 0)),
            pl.BlockSpec((1, d), lambda i: (0, 0)),
            pl.BlockSpec((1, d), lambda i: (0, 0)),
        ],
        out_shape=[
            jax.ShapeDtypeStruct((m, d), jnp.float32),
            jax.ShapeDtypeStruct((1, d), jnp.float32),
            jax.ShapeDtypeStruct((1, d), jnp.float32),
        ],
        scratch_shapes=[pltpu.VMEM((2, d), jnp.float32)],
    )(gp, xyzrep, wxn, bias, gam, bet)


def _pairmm_body(y_ref, sp_ref, tp_ref, w_ref, b_ref, gam_ref, bet_ref,
                 o_ref, s_ref, t_ref, acc, *, nsteps, count):
    i = pl.program_id(0)

    @pl.when(i == 0)
    def _init():
        acc[...] = jnp.zeros_like(acc)

    sp = sp_ref[...]
    tp = tp_ref[...]
    s2 = jnp.concatenate([sp, sp], axis=1)
    t2 = jnp.concatenate([tp, tp], axis=1)
    z = jnp.maximum(y_ref[...] * s2 + t2, 0.0)
    y = jnp.dot(z, w_ref[...], preferred_element_type=jnp.float32) + b_ref[...]
    o_ref[...] = y
    acc[0:1, :] = acc[0:1, :] + jnp.sum(y, axis=0, keepdims=True)
    acc[1:2, :] = acc[1:2, :] + jnp.sum(y * y, axis=0, keepdims=True)

    @pl.when(i == nsteps - 1)
    def _fin():
        inv = jnp.float32(1.0 / count)
        mean = acc[0:1, :] * inv
        var = acc[1:2, :] * inv - mean * mean
        sc = gam_ref[...] / jnp.sqrt(var + _EPS)
        s_ref[...] = sc
        t_ref[...] = bet_ref[...] - mean * sc


def _pairmm_call(yp, sp, tp, w, bias, gam, bet, rb=1024):
    m, twod = yp.shape
    d = twod // 2
    nsteps = m // rb
    return pl.pallas_call(
        functools.partial(_pairmm_body, nsteps=nsteps, count=m),
        grid=(nsteps,),
        in_specs=[
            pl.BlockSpec((rb, twod), lambda i: (i, 0)),
            pl.BlockSpec((1, d), lambda i: (0, 0)),
            pl.BlockSpec((1, d), lambda i: (0, 0)),
            pl.BlockSpec((twod, d), lambda i: (0, 0)),
            pl.BlockSpec((1, d), lambda i: (0, 0)),
            pl.BlockSpec((1, d), lambda i: (0, 0)),
            pl.BlockSpec((1, d), lambda i: (0, 0)),
        ],
        out_specs=[
            pl.BlockSpec((rb, d), lambda i: (i, 0)),
            pl.BlockSpec((1, d), lambda i: (0, 0)),
            pl.BlockSpec((1, d), lambda i: (0, 0)),
        ],
        out_shape=[
            jax.ShapeDtypeStruct((m, d), jnp.float32),
            jax.ShapeDtypeStruct((1, d), jnp.float32),
            jax.ShapeDtypeStruct((1, d), jnp.float32),
        ],
        scratch_shapes=[pltpu.VMEM((2, d), jnp.float32)],
    )(yp, sp, tp, w, bias, gam, bet)


def _final_body(y_ref, s_ref, t_ref, p_ref, o_ref):
    o_ref[...] = jnp.maximum(
        y_ref[...] * s_ref[...] + t_ref[...] + p_ref[...], 0.0)


def _final_call(y, s, t, pts, rb=1024):
    m, d = y.shape
    return pl.pallas_call(
        _final_body,
        grid=(m // rb,),
        in_specs=[
            pl.BlockSpec((rb, d), lambda i: (i, 0)),
            pl.BlockSpec((1, d), lambda i: (0, 0)),
            pl.BlockSpec((1, d), lambda i: (0, 0)),
            pl.BlockSpec((rb, d), lambda i: (i, 0)),
        ],
        out_specs=pl.BlockSpec((rb, d), lambda i: (i, 0)),
        out_shape=jax.ShapeDtypeStruct((m, d), jnp.float32),
    )(y, s, t, pts)


def _tap_weights(w):
    wx = jnp.stack([w[:, :3, 0, 0].T, w[:, :3, 0, 1].T])
    wx8 = jnp.pad(wx, ((0, 0), (0, 5), (0, 0)))
    wf = jnp.stack([w[:, 3:, 0, 0].T, w[:, 3:, 0, 1].T])
    return wx8, wf


def _pair_weight(w):
    return jnp.concatenate([w[:, :, 0, 0].T, w[:, :, 0, 1].T], axis=0)


def kernel(xyz, points, conv1_params, conv2_params):
    b, n, _ = xyz.shape
    c = points.shape[-1]
    m = b * n

    xyz8 = jnp.pad(xyz, ((0, 0), (0, 0), (0, 5)))
    xyzt8 = jnp.transpose(xyz8, (0, 2, 1))
    idxg = _select_call(xyz8, xyzt8)
    idx_flat = jnp.transpose(idxg, (0, 2, 1)).reshape(-1)

    xyz8f = xyz8.reshape(m, 8)
    pts = points.reshape(m, c)
    xyzrep = jnp.broadcast_to(
        xyz8.reshape(b, n, 1, 8), (b, n, 4, 8)).reshape(4 * m, 8)

    (w1, b1, g1, be1), (w2, b2, g2, be2), (w3, b3, g3, be3) = conv1_params
    (v1, c1, h1, ce1), (v2, c2, h2, ce2), (v3, c3, h3, ce3) = conv2_params

    wx8, wf = _tap_weights(w1)
    tt = _table_call(pts, xyz8f, wf, wx8)
    o = tt.shape[-1]
    gg = _sc_gather_call(tt.reshape(2 * m, o), idx_flat)
    wxn = -(wx8[0] + wx8[1])
    y1, s1, t1 = _pairsum_call(gg.reshape(4 * m, 2 * o), xyzrep, wxn,
                               b1[None], g1[None], be1[None])
    y2, s2, t2 = _pairmm_call(y1.reshape(2 * m, 2 * o), s1, t1,
                              _pair_weight(w2), b2[None], g2[None], be2[None])
    y3, s3, t3 = _pairmm_call(y2.reshape(m, 2 * o), s2, t2,
                              _pair_weight(w3), b3[None], g3[None], be3[None])

    vx8, vf = _tap_weights(v1)
    uu = _table_affine_call(y3, s3, t3, xyz8f, vf, vx8)
    hh = _sc_gather_call(uu.reshape(2 * m, o), idx_flat)
    vxn = -(vx8[0] + vx8[1])
    y4, s4, t4 = _pairsum_call(hh.reshape(4 * m, 2 * o), xyzrep, vxn,
                               c1[None], h1[None], ce1[None])
    y5, s5, t5 = _pairmm_call(y4.reshape(2 * m, 2 * o), s4, t4,
                              _pair_weight(v2), c2[None], h2[None], ce2[None])
    y6, s6, t6 = _pairmm_call(y5.reshape(m, 2 * o), s5, t5,
                              _pair_weight(v3), c3[None], h3[None], ce3[None])

    out = _final_call(y6, s6, t6, pts)
    return (xyz, out.reshape(b, n, o))


# speedup vs baseline: 7.0070x; 1.5439x over previous
import functools

import jax
import jax.numpy as jnp
from jax import lax
from jax.experimental import pallas as pl
from jax.experimental.pallas import tpu as pltpu
from jax.experimental.pallas import tpu_sc as plsc

_RADIUS = 0.2
_R2 = _RADIUS * _RADIUS
_EPS = 1e-5


def _select_body(xyz8_ref, xyzt8_ref, out_ref, *, n_total, bn_total, cb):
    b = pl.program_id(0)
    nb = pl.program_id(1)
    ctr = xyz8_ref[0]
    xt = xyzt8_ref[0]
    colid = lax.broadcasted_iota(jnp.int32, (cb, n_total), 1)
    rowid = lax.broadcasted_iota(jnp.int32, (cb, n_total), 0) + nb * cb
    d0 = xt[0:1, :] - ctr[:, 0:1]
    d1 = xt[1:2, :] - ctr[:, 1:2]
    d2 = xt[2:3, :] - ctr[:, 2:3]
    dist = (d0 * d0 + d1 * d1) + d2 * d2
    r2 = jnp.float32(_R2)
    big = jnp.float32(1e10)
    base = (dist > 1e-10) & (dist < r2)
    eye = colid == rowid
    sub = ((jnp.where(d0 >= 0.0, jnp.int32(4), jnp.int32(0))
            + jnp.where(d1 >= 0.0, jnp.int32(2), jnp.int32(0)))
           + jnp.where(d2 >= 0.0, jnp.int32(1), jnp.int32(0)))
    val_base = jnp.where(base, dist, big)
    val_base = jnp.where(eye, r2, val_base)
    for i in range(8):
        sel = (sub == i) | eye
        val = jnp.where(sel, val_base, big)
        mn = jnp.min(val, axis=1, keepdims=True)
        cand = jnp.where(val == mn, colid, jnp.int32(n_total))
        pj = jnp.min(cand, axis=1)
        out_ref[(i >> 1) + 4 * (i & 1), :] = (
            pj + (b * n_total + (i & 1) * bn_total))


def _select_call(xyz8, xyzt8, cb=256):
    b, n, _ = xyz8.shape
    nb = n // cb
    return pl.pallas_call(
        functools.partial(_select_body, n_total=n, bn_total=b * n, cb=cb),
        grid=(b, nb),
        in_specs=[
            pl.BlockSpec((1, cb, 8), lambda bi, ni: (bi, ni, 0)),
            pl.BlockSpec((1, 8, n), lambda bi, ni: (bi, 0, 0)),
        ],
        out_specs=pl.BlockSpec((8, cb), lambda bi, ni: (0, bi * nb + ni)),
        out_shape=jax.ShapeDtypeStruct((8, b * n), jnp.int32),
    )(xyz8, xyzt8)


def _table_body(feat_ref, xyz8_ref, wf_ref, wx_ref, out_ref):
    out_ref[0] = (jnp.dot(feat_ref[...], wf_ref[0],
                          preferred_element_type=jnp.float32)
                  + jnp.dot(xyz8_ref[...], wx_ref[0],
                            preferred_element_type=jnp.float32))


def _table_affine_body(y_ref, s_ref, t_ref, xyz8_ref, wf_ref, wx_ref, out_ref):
    z = jnp.maximum(y_ref[...] * s_ref[...] + t_ref[...], 0.0)
    out_ref[0] = (jnp.dot(z, wf_ref[0], preferred_element_type=jnp.float32)
                  + jnp.dot(xyz8_ref[...], wx_ref[0],
                            preferred_element_type=jnp.float32))


def _table_call(feat, xyz8f, wf, wx, rb=512):
    m, c = feat.shape
    o = wf.shape[-1]
    return pl.pallas_call(
        _table_body,
        grid=(2, m // rb),
        in_specs=[
            pl.BlockSpec((rb, c), lambda p, i: (i, 0)),
            pl.BlockSpec((rb, 8), lambda p, i: (i, 0)),
            pl.BlockSpec((1, c, o), lambda p, i: (p, 0, 0)),
            pl.BlockSpec((1, 8, o), lambda p, i: (p, 0, 0)),
        ],
        out_specs=pl.BlockSpec((1, rb, o), lambda p, i: (p, i, 0)),
        out_shape=jax.ShapeDtypeStruct((2, m, o), jnp.float32),
    )(feat, xyz8f, wf, wx)


def _table_affine_call(y, s, t, xyz8f, wf, wx, rb=512):
    m, c = y.shape
    o = wf.shape[-1]
    return pl.pallas_call(
        _table_affine_body,
        grid=(2, m // rb),
        in_specs=[
            pl.BlockSpec((rb, c), lambda p, i: (i, 0)),
            pl.BlockSpec((1, c), lambda p, i: (0, 0)),
            pl.BlockSpec((1, c), lambda p, i: (0, 0)),
            pl.BlockSpec((rb, 8), lambda p, i: (i, 0)),
            pl.BlockSpec((1, c, o), lambda p, i: (p, 0, 0)),
            pl.BlockSpec((1, 8, o), lambda p, i: (p, 0, 0)),
        ],
        out_specs=pl.BlockSpec((1, rb, o), lambda p, i: (p, i, 0)),
        out_shape=jax.ShapeDtypeStruct((2, m, o), jnp.float32),
    )(y, s, t, xyz8f, wf, wx)


def _sc_gather_call(table, idx_flat):
    v, d = table.shape
    (r,) = idx_flat.shape
    nc, ns = 2, 16
    nw = nc * ns
    per_w = r // nw
    ch = 128
    nchunk = per_w // ch
    mesh = plsc.VectorSubcoreMesh(core_axis_name="c", subcore_axis_name="s")

    @functools.partial(
        pl.kernel,
        out_type=jax.ShapeDtypeStruct((r, d), jnp.float32),
        mesh=mesh,
        scratch_types=[
            pltpu.VMEM((ch,), jnp.int32),
            pltpu.VMEM((ch,), jnp.int32),
            pltpu.VMEM((ch, d), jnp.float32),
            pltpu.VMEM((ch, d), jnp.float32),
            pltpu.SemaphoreType.DMA,
            pltpu.SemaphoreType.DMA,
            pltpu.SemaphoreType.DMA,
            pltpu.SemaphoreType.DMA,
        ],
    )
    def gk(table_hbm, idx_hbm, out_hbm, idx0, idx1, row0, row1,
           g0, g1, w0, w1):
        wid = lax.axis_index("s") * nc + lax.axis_index("c")
        base = wid * per_w
        idx_v = (idx0, idx1)
        row_v = (row0, row1)
        gsem = (g0, g1)
        wsem = (w0, w1)
        gh = [None] * nchunk
        wh = [None] * nchunk
        pltpu.sync_copy(idx_hbm.at[pl.ds(base, ch)], idx0)
        gh[0] = pltpu.async_copy(table_hbm.at[idx0], row0, g0)
        for i in range(nchunk):
            cur = i % 2
            nxt = 1 - cur
            if i + 1 < nchunk:
                pltpu.sync_copy(
                    idx_hbm.at[pl.ds(base + (i + 1) * ch, ch)], idx_v[nxt])
            gh[i].wait()
            if i > 0:
                wh[i - 1].wait()
            wh[i] = pltpu.async_copy(
                row_v[cur], out_hbm.at[pl.ds(base + i * ch, ch)], wsem[cur])
            if i + 1 < nchunk:
                gh[i + 1] = pltpu.async_copy(
                    table_hbm.at[idx_v[nxt]], row_v[nxt], gsem[nxt])
        wh[nchunk - 1].wait()

    return gk(table, idx_flat)


def _pairsum_body(ga_ref, gb_ref, xt_ref, wxn_ref, bias_ref, gam_ref, bet_ref,
                  y_ref, s_ref, t_ref, acc, *, kdim, nsteps, count):
    k = pl.program_id(0)
    i = pl.program_id(1)

    @pl.when((k == 0) & (i == 0))
    def _init():
        acc[...] = jnp.zeros_like(acc)

    cc = lax.dot_general(xt_ref[...], wxn_ref[...],
                         (((0,), (0,)), ((), ())),
                         preferred_element_type=jnp.float32)
    y = ga_ref[0, 0] + gb_ref[0, 0] + cc + bias_ref[...]
    y_ref[0] = y
    acc[0:1, :] = acc[0:1, :] + jnp.sum(y, axis=0, keepdims=True)
    acc[1:2, :] = acc[1:2, :] + jnp.sum(y * y, axis=0, keepdims=True)

    @pl.when((k == kdim - 1) & (i == nsteps - 1))
    def _fin():
        inv = jnp.float32(1.0 / count)
        mean = acc[0:1, :] * inv
        var = acc[1:2, :] * inv - mean * mean
        sc = gam_ref[...] / jnp.sqrt(var + _EPS)
        s_ref[...] = sc
        t_ref[...] = bet_ref[...] - mean * sc


def _pairsum_call(g4, xt, wxn, bias, gam, bet, rb=1024):
    _, kdim, m, d = g4.shape
    nsteps = m // rb
    count = kdim * m
    y, s, t = pl.pallas_call(
        functools.partial(_pairsum_body, kdim=kdim, nsteps=nsteps,
                          count=count),
        grid=(kdim, nsteps),
        in_specs=[
            pl.BlockSpec((1, 1, rb, d), lambda k, i: (0, k, i, 0)),
            pl.BlockSpec((1, 1, rb, d), lambda k, i: (1, k, i, 0)),
            pl.BlockSpec((8, rb), lambda k, i: (0, i)),
            pl.BlockSpec((8, d), lambda k, i: (0, 0)),
            pl.BlockSpec((1, d), lambda k, i: (0, 0)),
            pl.BlockSpec((1, d), lambda k, i: (0, 0)),
            pl.BlockSpec((1, d), lambda k, i: (0, 0)),
        ],
        out_specs=[
            pl.BlockSpec((1, rb, d), lambda k, i: (k, i, 0)),
            pl.BlockSpec((1, d), lambda k, i: (0, 0)),
            pl.BlockSpec((1, d), lambda k, i: (0, 0)),
        ],
        out_shape=[
            jax.ShapeDtypeStruct((kdim, m, d), jnp.float32),
            jax.ShapeDtypeStruct((1, d), jnp.float32),
            jax.ShapeDtypeStruct((1, d), jnp.float32),
        ],
        scratch_shapes=[pltpu.VMEM((2, d), jnp.float32)],
    )(g4, g4, xt, wxn, bias, gam, bet)
    return y, s, t


def _dualmm_body(ya_ref, yb_ref, sp_ref, tp_ref, wa_ref, wb_ref, b_ref,
                 gam_ref, bet_ref, o_ref, s_ref, t_ref, acc,
                 *, jdim, nsteps, count):
    j = pl.program_id(0)
    i = pl.program_id(1)

    @pl.when((j == 0) & (i == 0))
    def _init():
        acc[...] = jnp.zeros_like(acc)

    sp = sp_ref[...]
    tp = tp_ref[...]
    za = jnp.maximum(ya_ref[0] * sp + tp, 0.0)
    zb = jnp.maximum(yb_ref[0] * sp + tp, 0.0)
    y = (jnp.dot(za, wa_ref[0], preferred_element_type=jnp.float32)
         + jnp.dot(zb, wb_ref[0], preferred_element_type=jnp.float32)
         + b_ref[...])
    o_ref[0] = y
    acc[0:1, :] = acc[0:1, :] + jnp.sum(y, axis=0, keepdims=True)
    acc[1:2, :] = acc[1:2, :] + jnp.sum(y * y, axis=0, keepdims=True)

    @pl.when((j == jdim - 1) & (i == nsteps - 1))
    def _fin():
        inv = jnp.float32(1.0 / count)
        mean = acc[0:1, :] * inv
        var = acc[1:2, :] * inv - mean * mean
        sc = gam_ref[...] / jnp.sqrt(var + _EPS)
        s_ref[...] = sc
        t_ref[...] = bet_ref[...] - mean * sc


def _dualmm_call(yk, sp, tp, w2, bias, gam, bet, rb=1024):
    kin, m, d = yk.shape
    jdim = kin // 2
    nsteps = m // rb
    count = jdim * m
    return pl.pallas_call(
        functools.partial(_dualmm_body, jdim=jdim, nsteps=nsteps,
                          count=count),
        grid=(jdim, nsteps),
        in_specs=[
            pl.BlockSpec((1, rb, d), lambda j, i: (2 * j, i, 0)),
            pl.BlockSpec((1, rb, d), lambda j, i: (2 * j + 1, i, 0)),
            pl.BlockSpec((1, d), lambda j, i: (0, 0)),
            pl.BlockSpec((1, d), lambda j, i: (0, 0)),
            pl.BlockSpec((1, d, d), lambda j, i: (0, 0, 0)),
            pl.BlockSpec((1, d, d), lambda j, i: (1, 0, 0)),
            pl.BlockSpec((1, d), lambda j, i: (0, 0)),
            pl.BlockSpec((1, d), lambda j, i: (0, 0)),
            pl.BlockSpec((1, d), lambda j, i: (0, 0)),
        ],
        out_specs=[
            pl.BlockSpec((1, rb, d), lambda j, i: (j, i, 0)),
            pl.BlockSpec((1, d), lambda j, i: (0, 0)),
            pl.BlockSpec((1, d), lambda j, i: (0, 0)),
        ],
        out_shape=[
            jax.ShapeDtypeStruct((jdim, m, d), jnp.float32),
            jax.ShapeDtypeStruct((1, d), jnp.float32),
            jax.ShapeDtypeStruct((1, d), jnp.float32),
        ],
        scratch_shapes=[pltpu.VMEM((2, d), jnp.float32)],
    )(yk, yk, sp, tp, w2, w2, bias, gam, bet)


def _final_body(y_ref, s_ref, t_ref, p_ref, o_ref):
    o_ref[...] = jnp.maximum(
        y_ref[...] * s_ref[...] + t_ref[...] + p_ref[...], 0.0)


def _final_call(y, s, t, pts, rb=1024):
    m, d = y.shape
    return pl.pallas_call(
        _final_body,
        grid=(m // rb,),
        in_specs=[
            pl.BlockSpec((rb, d), lambda i: (i, 0)),
            pl.BlockSpec((1, d), lambda i: (0, 0)),
            pl.BlockSpec((1, d), lambda i: (0, 0)),
            pl.BlockSpec((rb, d), lambda i: (i, 0)),
        ],
        out_specs=pl.BlockSpec((rb, d), lambda i: (i, 0)),
        out_shape=jax.ShapeDtypeStruct((m, d), jnp.float32),
    )(y, s, t, pts)


def _tap_weights(w):
    wx = jnp.stack([w[:, :3, 0, 0].T, w[:, :3, 0, 1].T])
    wx8 = jnp.pad(wx, ((0, 0), (0, 5), (0, 0)))
    wf = jnp.stack([w[:, 3:, 0, 0].T, w[:, 3:, 0, 1].T])
    return wx8, wf


def _pair_weight(w):
    return jnp.stack([w[:, :, 0, 0].T, w[:, :, 0, 1].T])


def kernel(xyz, points, conv1_params, conv2_params):
    b, n, _ = xyz.shape
    c = points.shape[-1]
    m = b * n

    xyz8 = jnp.pad(xyz, ((0, 0), (0, 0), (0, 5)))
    xyzt8 = jnp.transpose(xyz8, (0, 2, 1))
    idx_flat = _select_call(xyz8, xyzt8).reshape(-1)

    xyz8f = xyz8.reshape(m, 8)
    xt = jnp.transpose(xyz8, (2, 0, 1)).reshape(8, m)
    pts = points.reshape(m, c)

    (w1, b1, g1, be1), (w2, b2, g2, be2), (w3, b3, g3, be3) = conv1_params
    (v1, c1, h1, ce1), (v2, c2, h2, ce2), (v3, c3, h3, ce3) = conv2_params

    wx8, wf = _tap_weights(w1)
    tt = _table_call(pts, xyz8f, wf, wx8)
    o = tt.shape[-1]
    gg = _sc_gather_call(tt.reshape(2 * m, o), idx_flat)
    wxn = -(wx8[0] + wx8[1])
    y1, s1, t1 = _pairsum_call(gg.reshape(2, 4, m, o), xt, wxn,
                               b1[None], g1[None], be1[None])
    y2, s2, t2 = _dualmm_call(y1, s1, t1, _pair_weight(w2),
                              b2[None], g2[None], be2[None])
    y3, s3, t3 = _dualmm_call(y2, s2, t2, _pair_weight(w3),
                              b3[None], g3[None], be3[None])

    vx8, vf = _tap_weights(v1)
    uu = _table_affine_call(y3.reshape(m, o), s3, t3, xyz8f, vf, vx8)
    hh = _sc_gather_call(uu.reshape(2 * m, o), idx_flat)
    vxn = -(vx8[0] + vx8[1])
    y4, s4, t4 = _pairsum_call(hh.reshape(2, 4, m, o), xt, vxn,
                               c1[None], h1[None], ce1[None])
    y5, s5, t5 = _dualmm_call(y4, s4, t4, _pair_weight(v2),
                              c2[None], h2[None], ce2[None])
    y6, s6, t6 = _dualmm_call(y5, s5, t5, _pair_weight(v3),
                              c3[None], h3[None], ce3[None])

    out = _final_call(y6.reshape(m, o), s6, t6, pts)
    return (xyz, out.reshape(b, n, o))


# speedup vs baseline: 7.8898x; 1.1260x over previous
import functools

import jax
import jax.numpy as jnp
from jax import lax
from jax.experimental import pallas as pl
from jax.experimental.pallas import tpu as pltpu
from jax.experimental.pallas import tpu_sc as plsc

_RADIUS = 0.2
_R2 = _RADIUS * _RADIUS
_EPS = 1e-5


def _select_body(xyz8_ref, xyzt8_ref, out_ref, *, n_total, bn_total, cb):
    b = pl.program_id(0)
    nb = pl.program_id(1)
    ctr = xyz8_ref[0]
    xt = xyzt8_ref[0]
    colid = lax.broadcasted_iota(jnp.int32, (cb, n_total), 1)
    rowid = lax.broadcasted_iota(jnp.int32, (cb, n_total), 0) + nb * cb
    d0 = xt[0:1, :] - ctr[:, 0:1]
    d1 = xt[1:2, :] - ctr[:, 1:2]
    d2 = xt[2:3, :] - ctr[:, 2:3]
    dist = (d0 * d0 + d1 * d1) + d2 * d2
    r2 = jnp.float32(_R2)
    big = jnp.float32(1e10)
    base = (dist > 1e-10) & (dist < r2)
    eye = colid == rowid
    sub = ((jnp.where(d0 >= 0.0, jnp.int32(4), jnp.int32(0))
            + jnp.where(d1 >= 0.0, jnp.int32(2), jnp.int32(0)))
           + jnp.where(d2 >= 0.0, jnp.int32(1), jnp.int32(0)))
    val_base = jnp.where(base, dist, big)
    val_base = jnp.where(eye, r2, val_base)
    for i in range(8):
        sel = (sub == i) | eye
        val = jnp.where(sel, val_base, big)
        mn = jnp.min(val, axis=1, keepdims=True)
        cand = jnp.where(val == mn, colid, jnp.int32(n_total))
        pj = jnp.min(cand, axis=1)
        out_ref[(i >> 1) + 4 * (i & 1), :] = (
            pj + (b * n_total + (i & 1) * bn_total))


def _select_call(xyz8, xyzt8, cb=512):
    b, n, _ = xyz8.shape
    nb = n // cb
    return pl.pallas_call(
        functools.partial(_select_body, n_total=n, bn_total=b * n, cb=cb),
        grid=(b, nb),
        in_specs=[
            pl.BlockSpec((1, cb, 8), lambda bi, ni: (bi, ni, 0)),
            pl.BlockSpec((1, 8, n), lambda bi, ni: (bi, 0, 0)),
        ],
        out_specs=pl.BlockSpec((8, cb), lambda bi, ni: (0, bi * nb + ni)),
        out_shape=jax.ShapeDtypeStruct((8, b * n), jnp.int32),
    )(xyz8, xyzt8)


def _pack_bf16(y):
    h = y.shape[-1] // 2
    yb = y.astype(jnp.bfloat16)
    lo = lax.bitcast_convert_type(yb[:, :h], jnp.uint16).astype(jnp.uint32)
    hi = lax.bitcast_convert_type(yb[:, h:], jnp.uint16).astype(jnp.uint32)
    return lo | (hi << 16)


def _unpack_bf16(g):
    lo = lax.bitcast_convert_type(g.astype(jnp.uint16), jnp.bfloat16)
    hi = lax.bitcast_convert_type((g >> 16).astype(jnp.uint16), jnp.bfloat16)
    return jnp.concatenate([lo, hi], axis=-1).astype(jnp.float32)


def _table_body(feat_ref, xyz8_ref, wf_ref, wx_ref, out_ref):
    y = (jnp.dot(feat_ref[...], wf_ref[0],
                 preferred_element_type=jnp.float32)
         + jnp.dot(xyz8_ref[...], wx_ref[0],
                   preferred_element_type=jnp.float32))
    out_ref[0] = _pack_bf16(y) if out_ref.dtype == jnp.uint32 else y


def _table_affine_body(y_ref, s_ref, t_ref, xyz8_ref, wf_ref, wx_ref, out_ref):
    z = jnp.maximum(y_ref[...] * s_ref[...] + t_ref[...], 0.0)
    y = (jnp.dot(z, wf_ref[0], preferred_element_type=jnp.float32)
         + jnp.dot(xyz8_ref[...], wx_ref[0],
                   preferred_element_type=jnp.float32))
    out_ref[0] = _pack_bf16(y) if out_ref.dtype == jnp.uint32 else y


def _table_call(feat, xyz8f, wf, wx, rb=512, pack=True):
    m, c = feat.shape
    o = wf.shape[-1]
    oo = o // 2 if pack else o
    odt = jnp.uint32 if pack else jnp.float32
    return pl.pallas_call(
        _table_body,
        grid=(2, m // rb),
        in_specs=[
            pl.BlockSpec((rb, c), lambda p, i: (i, 0)),
            pl.BlockSpec((rb, 8), lambda p, i: (i, 0)),
            pl.BlockSpec((1, c, o), lambda p, i: (p, 0, 0)),
            pl.BlockSpec((1, 8, o), lambda p, i: (p, 0, 0)),
        ],
        out_specs=pl.BlockSpec((1, rb, oo), lambda p, i: (p, i, 0)),
        out_shape=jax.ShapeDtypeStruct((2, m, oo), odt),
    )(feat, xyz8f, wf, wx)


def _table_affine_call(y, s, t, xyz8f, wf, wx, rb=512, pack=True):
    m, c = y.shape
    o = wf.shape[-1]
    oo = o // 2 if pack else o
    odt = jnp.uint32 if pack else jnp.float32
    return pl.pallas_call(
        _table_affine_body,
        grid=(2, m // rb),
        in_specs=[
            pl.BlockSpec((rb, c), lambda p, i: (i, 0)),
            pl.BlockSpec((1, c), lambda p, i: (0, 0)),
            pl.BlockSpec((1, c), lambda p, i: (0, 0)),
            pl.BlockSpec((rb, 8), lambda p, i: (i, 0)),
            pl.BlockSpec((1, c, o), lambda p, i: (p, 0, 0)),
            pl.BlockSpec((1, 8, o), lambda p, i: (p, 0, 0)),
        ],
        out_specs=pl.BlockSpec((1, rb, oo), lambda p, i: (p, i, 0)),
        out_shape=jax.ShapeDtypeStruct((2, m, oo), odt),
    )(y, s, t, xyz8f, wf, wx)


def _sc_gather_call(table, idx_flat):
    v, d = table.shape
    dt = table.dtype
    (r,) = idx_flat.shape
    nc, ns = 2, 16
    nw = nc * ns
    per_w = r // nw
    ch = 128
    nchunk = per_w // ch
    mesh = plsc.VectorSubcoreMesh(core_axis_name="c", subcore_axis_name="s")

    @functools.partial(
        pl.kernel,
        out_type=jax.ShapeDtypeStruct((r, d), dt),
        mesh=mesh,
        scratch_types=[
            pltpu.VMEM((ch,), jnp.int32),
            pltpu.VMEM((ch,), jnp.int32),
            pltpu.VMEM((ch, d), dt),
            pltpu.VMEM((ch, d), dt),
            pltpu.SemaphoreType.DMA,
            pltpu.SemaphoreType.DMA,
            pltpu.SemaphoreType.DMA,
            pltpu.SemaphoreType.DMA,
        ],
    )
    def gk(table_hbm, idx_hbm, out_hbm, idx0, idx1, row0, row1,
           g0, g1, w0, w1):
        wid = lax.axis_index("s") * nc + lax.axis_index("c")
        base = wid * per_w
        idx_v = (idx0, idx1)
        row_v = (row0, row1)
        gsem = (g0, g1)
        wsem = (w0, w1)
        gh = [None] * nchunk
        wh = [None] * nchunk
        pltpu.sync_copy(idx_hbm.at[pl.ds(base, ch)], idx0)
        gh[0] = pltpu.async_copy(table_hbm.at[idx0], row0, g0)
        for i in range(nchunk):
            cur = i % 2
            nxt = 1 - cur
            if i + 1 < nchunk:
                pltpu.sync_copy(
                    idx_hbm.at[pl.ds(base + (i + 1) * ch, ch)], idx_v[nxt])
            gh[i].wait()
            if i > 0:
                wh[i - 1].wait()
            wh[i] = pltpu.async_copy(
                row_v[cur], out_hbm.at[pl.ds(base + i * ch, ch)], wsem[cur])
            if i + 1 < nchunk:
                gh[i + 1] = pltpu.async_copy(
                    table_hbm.at[idx_v[nxt]], row_v[nxt], gsem[nxt])
        wh[nchunk - 1].wait()

    return gk(table, idx_flat)


def _pairsum_body(ga_ref, gb_ref, xt_ref, wxn_ref, bias_ref, gam_ref, bet_ref,
                  y_ref, s_ref, t_ref, acc, *, kdim, nsteps, count):
    k = pl.program_id(0)
    i = pl.program_id(1)

    @pl.when((k == 0) & (i == 0))
    def _init():
        acc[...] = jnp.zeros_like(acc)

    cc = lax.dot_general(xt_ref[...], wxn_ref[...],
                         (((0,), (0,)), ((), ())),
                         preferred_element_type=jnp.float32)
    ga = ga_ref[0, 0]
    gb = gb_ref[0, 0]
    if ga.dtype == jnp.uint32:
        ga = _unpack_bf16(ga)
        gb = _unpack_bf16(gb)
    y = ga + gb + cc + bias_ref[...]
    y_ref[0] = y
    acc[0:1, :] = acc[0:1, :] + jnp.sum(y, axis=0, keepdims=True)
    acc[1:2, :] = acc[1:2, :] + jnp.sum(y * y, axis=0, keepdims=True)

    @pl.when((k == kdim - 1) & (i == nsteps - 1))
    def _fin():
        inv = jnp.float32(1.0 / count)
        mean = acc[0:1, :] * inv
        var = acc[1:2, :] * inv - mean * mean
        sc = gam_ref[...] / jnp.sqrt(var + _EPS)
        s_ref[...] = sc
        t_ref[...] = bet_ref[...] - mean * sc


def _pairsum_call(g4, xt, wxn, bias, gam, bet, rb=1024):
    _, kdim, m, gw = g4.shape
    d = bias.shape[-1]
    nsteps = m // rb
    count = kdim * m
    y, s, t = pl.pallas_call(
        functools.partial(_pairsum_body, kdim=kdim, nsteps=nsteps,
                          count=count),
        grid=(kdim, nsteps),
        in_specs=[
            pl.BlockSpec((1, 1, rb, gw), lambda k, i: (0, k, i, 0)),
            pl.BlockSpec((1, 1, rb, gw), lambda k, i: (1, k, i, 0)),
            pl.BlockSpec((8, rb), lambda k, i: (0, i)),
            pl.BlockSpec((8, d), lambda k, i: (0, 0)),
            pl.BlockSpec((1, d), lambda k, i: (0, 0)),
            pl.BlockSpec((1, d), lambda k, i: (0, 0)),
            pl.BlockSpec((1, d), lambda k, i: (0, 0)),
        ],
        out_specs=[
            pl.BlockSpec((1, rb, d), lambda k, i: (k, i, 0)),
            pl.BlockSpec((1, d), lambda k, i: (0, 0)),
            pl.BlockSpec((1, d), lambda k, i: (0, 0)),
        ],
        out_shape=[
            jax.ShapeDtypeStruct((kdim, m, d), jnp.float32),
            jax.ShapeDtypeStruct((1, d), jnp.float32),
            jax.ShapeDtypeStruct((1, d), jnp.float32),
        ],
        scratch_shapes=[pltpu.VMEM((2, d), jnp.float32)],
    )(g4, g4, xt, wxn, bias, gam, bet)
    return y, s, t


def _dualmm_body(ya_ref, yb_ref, sp_ref, tp_ref, wa_ref, wb_ref, b_ref,
                 gam_ref, bet_ref, o_ref, s_ref, t_ref, acc,
                 *, jdim, nsteps, count):
    j = pl.program_id(0)
    i = pl.program_id(1)

    @pl.when((j == 0) & (i == 0))
    def _init():
        acc[...] = jnp.zeros_like(acc)

    sp = sp_ref[...]
    tp = tp_ref[...]
    za = jnp.maximum(ya_ref[0] * sp + tp, 0.0)
    zb = jnp.maximum(yb_ref[0] * sp + tp, 0.0)
    y = (jnp.dot(za, wa_ref[0], preferred_element_type=jnp.float32)
         + jnp.dot(zb, wb_ref[0], preferred_element_type=jnp.float32)
         + b_ref[...])
    o_ref[0] = y
    acc[0:1, :] = acc[0:1, :] + jnp.sum(y, axis=0, keepdims=True)
    acc[1:2, :] = acc[1:2, :] + jnp.sum(y * y, axis=0, keepdims=True)

    @pl.when((j == jdim - 1) & (i == nsteps - 1))
    def _fin():
        inv = jnp.float32(1.0 / count)
        mean = acc[0:1, :] * inv
        var = acc[1:2, :] * inv - mean * mean
        sc = gam_ref[...] / jnp.sqrt(var + _EPS)
        s_ref[...] = sc
        t_ref[...] = bet_ref[...] - mean * sc


def _dualmm_call(yk, sp, tp, w2, bias, gam, bet, rb=1024):
    kin, m, d = yk.shape
    jdim = kin // 2
    nsteps = m // rb
    count = jdim * m
    return pl.pallas_call(
        functools.partial(_dualmm_body, jdim=jdim, nsteps=nsteps,
                          count=count),
        grid=(jdim, nsteps),
        in_specs=[
            pl.BlockSpec((1, rb, d), lambda j, i: (2 * j, i, 0)),
            pl.BlockSpec((1, rb, d), lambda j, i: (2 * j + 1, i, 0)),
            pl.BlockSpec((1, d), lambda j, i: (0, 0)),
            pl.BlockSpec((1, d), lambda j, i: (0, 0)),
            pl.BlockSpec((1, d, d), lambda j, i: (0, 0, 0)),
            pl.BlockSpec((1, d, d), lambda j, i: (1, 0, 0)),
            pl.BlockSpec((1, d), lambda j, i: (0, 0)),
            pl.BlockSpec((1, d), lambda j, i: (0, 0)),
            pl.BlockSpec((1, d), lambda j, i: (0, 0)),
        ],
        out_specs=[
            pl.BlockSpec((1, rb, d), lambda j, i: (j, i, 0)),
            pl.BlockSpec((1, d), lambda j, i: (0, 0)),
            pl.BlockSpec((1, d), lambda j, i: (0, 0)),
        ],
        out_shape=[
            jax.ShapeDtypeStruct((jdim, m, d), jnp.float32),
            jax.ShapeDtypeStruct((1, d), jnp.float32),
            jax.ShapeDtypeStruct((1, d), jnp.float32),
        ],
        scratch_shapes=[pltpu.VMEM((2, d), jnp.float32)],
    )(yk, yk, sp, tp, w2, w2, bias, gam, bet)


def _final_body(y_ref, s_ref, t_ref, p_ref, o_ref):
    o_ref[...] = jnp.maximum(
        y_ref[...] * s_ref[...] + t_ref[...] + p_ref[...], 0.0)


def _final_call(y, s, t, pts, rb=1024):
    m, d = y.shape
    return pl.pallas_call(
        _final_body,
        grid=(m // rb,),
        in_specs=[
            pl.BlockSpec((rb, d), lambda i: (i, 0)),
            pl.BlockSpec((1, d), lambda i: (0, 0)),
            pl.BlockSpec((1, d), lambda i: (0, 0)),
            pl.BlockSpec((rb, d), lambda i: (i, 0)),
        ],
        out_specs=pl.BlockSpec((rb, d), lambda i: (i, 0)),
        out_shape=jax.ShapeDtypeStruct((m, d), jnp.float32),
    )(y, s, t, pts)


def _tap_weights(w):
    wx = jnp.stack([w[:, :3, 0, 0].T, w[:, :3, 0, 1].T])
    wx8 = jnp.pad(wx, ((0, 0), (0, 5), (0, 0)))
    wf = jnp.stack([w[:, 3:, 0, 0].T, w[:, 3:, 0, 1].T])
    return wx8, wf


def _pair_weight(w):
    return jnp.stack([w[:, :, 0, 0].T, w[:, :, 0, 1].T])


def kernel(xyz, points, conv1_params, conv2_params):
    b, n, _ = xyz.shape
    c = points.shape[-1]
    m = b * n

    xyz8 = jnp.pad(xyz, ((0, 0), (0, 0), (0, 5)))
    xyzt8 = jnp.transpose(xyz8, (0, 2, 1))
    idx_flat = _select_call(xyz8, xyzt8).reshape(-1)

    xyz8f = xyz8.reshape(m, 8)
    xt = jnp.transpose(xyz8, (2, 0, 1)).reshape(8, m)
    pts = points.reshape(m, c)

    (w1, b1, g1, be1), (w2, b2, g2, be2), (w3, b3, g3, be3) = conv1_params
    (v1, c1, h1, ce1), (v2, c2, h2, ce2), (v3, c3, h3, ce3) = conv2_params

    wx8, wf = _tap_weights(w1)
    tt = _table_call(pts, xyz8f, wf, wx8)
    gw = tt.shape[-1]
    o = b1.shape[0]
    gg = _sc_gather_call(tt.reshape(2 * m, gw), idx_flat)
    wxn = -(wx8[0] + wx8[1])
    y1, s1, t1 = _pairsum_call(gg.reshape(2, 4, m, gw), xt, wxn,
                               b1[None], g1[None], be1[None])
    y2, s2, t2 = _dualmm_call(y1, s1, t1, _pair_weight(w2),
                              b2[None], g2[None], be2[None])
    y3, s3, t3 = _dualmm_call(y2, s2, t2, _pair_weight(w3),
                              b3[None], g3[None], be3[None])

    vx8, vf = _tap_weights(v1)
    uu = _table_affine_call(y3.reshape(m, o), s3, t3, xyz8f, vf, vx8)
    hh = _sc_gather_call(uu.reshape(2 * m, gw), idx_flat)
    vxn = -(vx8[0] + vx8[1])
    y4, s4, t4 = _pairsum_call(hh.reshape(2, 4, m, gw), xt, vxn,
                               c1[None], h1[None], ce1[None])
    y5, s5, t5 = _dualmm_call(y4, s4, t4, _pair_weight(v2),
                              c2[None], h2[None], ce2[None])
    y6, s6, t6 = _dualmm_call(y5, s5, t5, _pair_weight(v3),
                              c3[None], h3[None], ce3[None])

    out = _final_call(y6.reshape(m, o), s6, t6, pts)
    return (xyz, out.reshape(b, n, o))


# speedup vs baseline: 8.0901x; 1.0254x over previous
import functools

import jax
import jax.numpy as jnp
from jax import lax
from jax.experimental import pallas as pl
from jax.experimental.pallas import tpu as pltpu
from jax.experimental.pallas import tpu_sc as plsc

_RADIUS = 0.2
_R2 = _RADIUS * _RADIUS
_EPS = 1e-5


def _select_body(xyz8_ref, xyzt8_ref, out_ref, *, n_total, bn_total, cb):
    b = pl.program_id(0)
    nb = pl.program_id(1)
    ctr = xyz8_ref[0]
    xt = xyzt8_ref[0]
    colid = lax.broadcasted_iota(jnp.int32, (cb, n_total), 1)
    rowid = lax.broadcasted_iota(jnp.int32, (cb, n_total), 0) + nb * cb
    d0 = xt[0:1, :] - ctr[:, 0:1]
    d1 = xt[1:2, :] - ctr[:, 1:2]
    d2 = xt[2:3, :] - ctr[:, 2:3]
    dist = (d0 * d0 + d1 * d1) + d2 * d2
    r2 = jnp.float32(_R2)
    big = jnp.float32(1e10)
    base = (dist > 1e-10) & (dist < r2)
    eye = colid == rowid
    sub = ((jnp.where(d0 >= 0.0, jnp.int32(4), jnp.int32(0))
            + jnp.where(d1 >= 0.0, jnp.int32(2), jnp.int32(0)))
           + jnp.where(d2 >= 0.0, jnp.int32(1), jnp.int32(0)))
    val_base = jnp.where(base, dist, big)
    val_base = jnp.where(eye, r2, val_base)
    for i in range(8):
        sel = (sub == i) | eye
        val = jnp.where(sel, val_base, big)
        mn = jnp.min(val, axis=1, keepdims=True)
        cand = jnp.where(val == mn, colid, jnp.int32(n_total))
        pj = jnp.min(cand, axis=1)
        out_ref[(i >> 1) + 4 * (i & 1), :] = (
            pj + (b * n_total + (i & 1) * bn_total))


def _select_call(xyz8, xyzt8, cb=512):
    b, n, _ = xyz8.shape
    nb = n // cb
    return pl.pallas_call(
        functools.partial(_select_body, n_total=n, bn_total=b * n, cb=cb),
        grid=(b, nb),
        in_specs=[
            pl.BlockSpec((1, cb, 8), lambda bi, ni: (bi, ni, 0)),
            pl.BlockSpec((1, 8, n), lambda bi, ni: (bi, 0, 0)),
        ],
        out_specs=pl.BlockSpec((8, cb), lambda bi, ni: (0, bi * nb + ni)),
        out_shape=jax.ShapeDtypeStruct((8, b * n), jnp.int32),
    )(xyz8, xyzt8)


def _pack_bf16(y):
    h = y.shape[-1] // 2
    yb = y.astype(jnp.bfloat16)
    lo = lax.bitcast_convert_type(yb[:, :h], jnp.uint16).astype(jnp.uint32)
    hi = lax.bitcast_convert_type(yb[:, h:], jnp.uint16).astype(jnp.uint32)
    return lo | (hi << 16)


def _unpack_bf16(g):
    lo = lax.bitcast_convert_type(g.astype(jnp.uint16), jnp.bfloat16)
    hi = lax.bitcast_convert_type((g >> 16).astype(jnp.uint16), jnp.bfloat16)
    return jnp.concatenate([lo, hi], axis=-1).astype(jnp.float32)


def _table_body(feat_ref, xyz8_ref, wf_ref, wx_ref, out_ref):
    y = (jnp.dot(feat_ref[...], wf_ref[0],
                 preferred_element_type=jnp.float32)
         + jnp.dot(xyz8_ref[...], wx_ref[0],
                   preferred_element_type=jnp.float32))
    out_ref[0] = _pack_bf16(y) if out_ref.dtype == jnp.uint32 else y


def _table_affine_body(y_ref, s_ref, t_ref, xyz8_ref, wf_ref, wx_ref, out_ref):
    z = jnp.maximum(y_ref[...] * s_ref[...] + t_ref[...], 0.0)
    y = (jnp.dot(z, wf_ref[0], preferred_element_type=jnp.float32)
         + jnp.dot(xyz8_ref[...], wx_ref[0],
                   preferred_element_type=jnp.float32))
    out_ref[0] = _pack_bf16(y) if out_ref.dtype == jnp.uint32 else y


def _table_call(feat, xyz8f, wf, wx, rb=512, pack=True):
    m, c = feat.shape
    o = wf.shape[-1]
    oo = o // 2 if pack else o
    odt = jnp.uint32 if pack else jnp.float32
    return pl.pallas_call(
        _table_body,
        grid=(2, m // rb),
        in_specs=[
            pl.BlockSpec((rb, c), lambda p, i: (i, 0)),
            pl.BlockSpec((rb, 8), lambda p, i: (i, 0)),
            pl.BlockSpec((1, c, o), lambda p, i: (p, 0, 0)),
            pl.BlockSpec((1, 8, o), lambda p, i: (p, 0, 0)),
        ],
        out_specs=pl.BlockSpec((1, rb, oo), lambda p, i: (p, i, 0)),
        out_shape=jax.ShapeDtypeStruct((2, m, oo), odt),
    )(feat, xyz8f, wf, wx)


def _table_affine_call(y, s, t, xyz8f, wf, wx, rb=512, pack=True):
    m, c = y.shape
    o = wf.shape[-1]
    oo = o // 2 if pack else o
    odt = jnp.uint32 if pack else jnp.float32
    return pl.pallas_call(
        _table_affine_body,
        grid=(2, m // rb),
        in_specs=[
            pl.BlockSpec((rb, c), lambda p, i: (i, 0)),
            pl.BlockSpec((1, c), lambda p, i: (0, 0)),
            pl.BlockSpec((1, c), lambda p, i: (0, 0)),
            pl.BlockSpec((rb, 8), lambda p, i: (i, 0)),
            pl.BlockSpec((1, c, o), lambda p, i: (p, 0, 0)),
            pl.BlockSpec((1, 8, o), lambda p, i: (p, 0, 0)),
        ],
        out_specs=pl.BlockSpec((1, rb, oo), lambda p, i: (p, i, 0)),
        out_shape=jax.ShapeDtypeStruct((2, m, oo), odt),
    )(y, s, t, xyz8f, wf, wx)


def _sc_gather_call(table, idx_flat):
    v, d = table.shape
    dt = table.dtype
    (r,) = idx_flat.shape
    nc, ns = 2, 16
    nw = nc * ns
    per_w = r // nw
    ch = 128
    nchunk = per_w // ch
    mesh = plsc.VectorSubcoreMesh(core_axis_name="c", subcore_axis_name="s")

    nbuf = 4

    @functools.partial(
        pl.kernel,
        out_type=jax.ShapeDtypeStruct((r, d), dt),
        mesh=mesh,
        scratch_types=(
            [pltpu.VMEM((per_w,), jnp.int32)]
            + [pltpu.VMEM((ch, d), dt) for _ in range(nbuf)]
            + [pltpu.SemaphoreType.DMA for _ in range(2 * nbuf)]
        ),
    )
    def gk(table_hbm, idx_hbm, out_hbm, idx_v, *bufs):
        row_v = bufs[:nbuf]
        gsem = bufs[nbuf:2 * nbuf]
        wsem = bufs[2 * nbuf:]
        wid = lax.axis_index("s") * nc + lax.axis_index("c")
        base = wid * per_w
        pltpu.sync_copy(idx_hbm.at[pl.ds(base, per_w)], idx_v)
        gh = [None] * nchunk
        wh = [None] * nchunk
        for i in range(min(nbuf, nchunk)):
            gh[i] = pltpu.async_copy(
                table_hbm.at[idx_v.at[pl.ds(i * ch, ch)]], row_v[i], gsem[i])
        for i in range(nchunk):
            cur = i % nbuf
            gh[i].wait()
            wh[i] = pltpu.async_copy(
                row_v[cur], out_hbm.at[pl.ds(base + i * ch, ch)], wsem[cur])
            if i + nbuf < nchunk:
                wh[i].wait()
                gh[i + nbuf] = pltpu.async_copy(
                    table_hbm.at[idx_v.at[pl.ds((i + nbuf) * ch, ch)]],
                    row_v[cur], gsem[cur])
        for i in range(max(0, nchunk - nbuf), nchunk):
            wh[i].wait()

    return gk(table, idx_flat)


def _pairsum_body(ga_ref, gb_ref, xt_ref, wxn_ref, bias_ref, gam_ref, bet_ref,
                  y_ref, s_ref, t_ref, acc, *, kdim, nsteps, count):
    k = pl.program_id(0)
    i = pl.program_id(1)

    @pl.when((k == 0) & (i == 0))
    def _init():
        acc[...] = jnp.zeros_like(acc)

    cc = lax.dot_general(xt_ref[...], wxn_ref[...],
                         (((0,), (0,)), ((), ())),
                         preferred_element_type=jnp.float32)
    ga = ga_ref[0, 0]
    gb = gb_ref[0, 0]
    if ga.dtype == jnp.uint32:
        ga = _unpack_bf16(ga)
        gb = _unpack_bf16(gb)
    y = ga + gb + cc + bias_ref[...]
    y_ref[0] = y
    acc[0:1, :] = acc[0:1, :] + jnp.sum(y, axis=0, keepdims=True)
    acc[1:2, :] = acc[1:2, :] + jnp.sum(y * y, axis=0, keepdims=True)

    @pl.when((k == kdim - 1) & (i == nsteps - 1))
    def _fin():
        inv = jnp.float32(1.0 / count)
        mean = acc[0:1, :] * inv
        var = acc[1:2, :] * inv - mean * mean
        sc = gam_ref[...] / jnp.sqrt(var + _EPS)
        s_ref[...] = sc
        t_ref[...] = bet_ref[...] - mean * sc


def _pairsum_call(g4, xt, wxn, bias, gam, bet, rb=1024):
    _, kdim, m, gw = g4.shape
    d = bias.shape[-1]
    nsteps = m // rb
    count = kdim * m
    y, s, t = pl.pallas_call(
        functools.partial(_pairsum_body, kdim=kdim, nsteps=nsteps,
                          count=count),
        grid=(kdim, nsteps),
        in_specs=[
            pl.BlockSpec((1, 1, rb, gw), lambda k, i: (0, k, i, 0)),
            pl.BlockSpec((1, 1, rb, gw), lambda k, i: (1, k, i, 0)),
            pl.BlockSpec((8, rb), lambda k, i: (0, i)),
            pl.BlockSpec((8, d), lambda k, i: (0, 0)),
            pl.BlockSpec((1, d), lambda k, i: (0, 0)),
            pl.BlockSpec((1, d), lambda k, i: (0, 0)),
            pl.BlockSpec((1, d), lambda k, i: (0, 0)),
        ],
        out_specs=[
            pl.BlockSpec((1, rb, d), lambda k, i: (k, i, 0)),
            pl.BlockSpec((1, d), lambda k, i: (0, 0)),
            pl.BlockSpec((1, d), lambda k, i: (0, 0)),
        ],
        out_shape=[
            jax.ShapeDtypeStruct((kdim, m, d), jnp.float32),
            jax.ShapeDtypeStruct((1, d), jnp.float32),
            jax.ShapeDtypeStruct((1, d), jnp.float32),
        ],
        scratch_shapes=[pltpu.VMEM((2, d), jnp.float32)],
    )(g4, g4, xt, wxn, bias, gam, bet)
    return y, s, t


def _dualmm_body(ya_ref, yb_ref, sp_ref, tp_ref, wa_ref, wb_ref, b_ref,
                 gam_ref, bet_ref, o_ref, s_ref, t_ref, acc,
                 *, jdim, nsteps, count):
    j = pl.program_id(0)
    i = pl.program_id(1)

    @pl.when((j == 0) & (i == 0))
    def _init():
        acc[...] = jnp.zeros_like(acc)

    sp = sp_ref[...]
    tp = tp_ref[...]
    za = jnp.maximum(ya_ref[0] * sp + tp, 0.0)
    zb = jnp.maximum(yb_ref[0] * sp + tp, 0.0)
    y = (jnp.dot(za, wa_ref[0], preferred_element_type=jnp.float32)
         + jnp.dot(zb, wb_ref[0], preferred_element_type=jnp.float32)
         + b_ref[...])
    o_ref[0] = y
    acc[0:1, :] = acc[0:1, :] + jnp.sum(y, axis=0, keepdims=True)
    acc[1:2, :] = acc[1:2, :] + jnp.sum(y * y, axis=0, keepdims=True)

    @pl.when((j == jdim - 1) & (i == nsteps - 1))
    def _fin():
        inv = jnp.float32(1.0 / count)
        mean = acc[0:1, :] * inv
        var = acc[1:2, :] * inv - mean * mean
        sc = gam_ref[...] / jnp.sqrt(var + _EPS)
        s_ref[...] = sc
        t_ref[...] = bet_ref[...] - mean * sc


def _dualmm_call(yk, sp, tp, w2, bias, gam, bet, rb=1024):
    kin, m, d = yk.shape
    jdim = kin // 2
    nsteps = m // rb
    count = jdim * m
    return pl.pallas_call(
        functools.partial(_dualmm_body, jdim=jdim, nsteps=nsteps,
                          count=count),
        grid=(jdim, nsteps),
        in_specs=[
            pl.BlockSpec((1, rb, d), lambda j, i: (2 * j, i, 0)),
            pl.BlockSpec((1, rb, d), lambda j, i: (2 * j + 1, i, 0)),
            pl.BlockSpec((1, d), lambda j, i: (0, 0)),
            pl.BlockSpec((1, d), lambda j, i: (0, 0)),
            pl.BlockSpec((1, d, d), lambda j, i: (0, 0, 0)),
            pl.BlockSpec((1, d, d), lambda j, i: (1, 0, 0)),
            pl.BlockSpec((1, d), lambda j, i: (0, 0)),
            pl.BlockSpec((1, d), lambda j, i: (0, 0)),
            pl.BlockSpec((1, d), lambda j, i: (0, 0)),
        ],
        out_specs=[
            pl.BlockSpec((1, rb, d), lambda j, i: (j, i, 0)),
            pl.BlockSpec((1, d), lambda j, i: (0, 0)),
            pl.BlockSpec((1, d), lambda j, i: (0, 0)),
        ],
        out_shape=[
            jax.ShapeDtypeStruct((jdim, m, d), jnp.float32),
            jax.ShapeDtypeStruct((1, d), jnp.float32),
            jax.ShapeDtypeStruct((1, d), jnp.float32),
        ],
        scratch_shapes=[pltpu.VMEM((2, d), jnp.float32)],
    )(yk, yk, sp, tp, w2, w2, bias, gam, bet)


def _final_body(y_ref, s_ref, t_ref, p_ref, o_ref):
    o_ref[...] = jnp.maximum(
        y_ref[...] * s_ref[...] + t_ref[...] + p_ref[...], 0.0)


def _final_call(y, s, t, pts, rb=1024):
    m, d = y.shape
    return pl.pallas_call(
        _final_body,
        grid=(m // rb,),
        in_specs=[
            pl.BlockSpec((rb, d), lambda i: (i, 0)),
            pl.BlockSpec((1, d), lambda i: (0, 0)),
            pl.BlockSpec((1, d), lambda i: (0, 0)),
            pl.BlockSpec((rb, d), lambda i: (i, 0)),
        ],
        out_specs=pl.BlockSpec((rb, d), lambda i: (i, 0)),
        out_shape=jax.ShapeDtypeStruct((m, d), jnp.float32),
    )(y, s, t, pts)


def _tap_weights(w):
    wx = jnp.stack([w[:, :3, 0, 0].T, w[:, :3, 0, 1].T])
    wx8 = jnp.pad(wx, ((0, 0), (0, 5), (0, 0)))
    wf = jnp.stack([w[:, 3:, 0, 0].T, w[:, 3:, 0, 1].T])
    return wx8, wf


def _pair_weight(w):
    return jnp.stack([w[:, :, 0, 0].T, w[:, :, 0, 1].T])


def kernel(xyz, points, conv1_params, conv2_params):
    b, n, _ = xyz.shape
    c = points.shape[-1]
    m = b * n

    xyz8 = jnp.pad(xyz, ((0, 0), (0, 0), (0, 5)))
    xyzt8 = jnp.transpose(xyz8, (0, 2, 1))
    idx_flat = _select_call(xyz8, xyzt8).reshape(-1)

    xyz8f = xyz8.reshape(m, 8)
    xt = jnp.transpose(xyz8, (2, 0, 1)).reshape(8, m)
    pts = points.reshape(m, c)

    (w1, b1, g1, be1), (w2, b2, g2, be2), (w3, b3, g3, be3) = conv1_params
    (v1, c1, h1, ce1), (v2, c2, h2, ce2), (v3, c3, h3, ce3) = conv2_params

    wx8, wf = _tap_weights(w1)
    tt = _table_call(pts, xyz8f, wf, wx8)
    gw = tt.shape[-1]
    o = b1.shape[0]
    gg = _sc_gather_call(tt.reshape(2 * m, gw), idx_flat)
    wxn = -(wx8[0] + wx8[1])
    y1, s1, t1 = _pairsum_call(gg.reshape(2, 4, m, gw), xt, wxn,
                               b1[None], g1[None], be1[None])
    y2, s2, t2 = _dualmm_call(y1, s1, t1, _pair_weight(w2),
                              b2[None], g2[None], be2[None])
    y3, s3, t3 = _dualmm_call(y2, s2, t2, _pair_weight(w3),
                              b3[None], g3[None], be3[None])

    vx8, vf = _tap_weights(v1)
    uu = _table_affine_call(y3.reshape(m, o), s3, t3, xyz8f, vf, vx8)
    hh = _sc_gather_call(uu.reshape(2 * m, gw), idx_flat)
    vxn = -(vx8[0] + vx8[1])
    y4, s4, t4 = _pairsum_call(hh.reshape(2, 4, m, gw), xt, vxn,
                               c1[None], h1[None], ce1[None])
    y5, s5, t5 = _dualmm_call(y4, s4, t4, _pair_weight(v2),
                              c2[None], h2[None], ce2[None])
    y6, s6, t6 = _dualmm_call(y5, s5, t5, _pair_weight(v3),
                              c3[None], h3[None], ce3[None])

    out = _final_call(y6.reshape(m, o), s6, t6, pts)
    return (xyz, out.reshape(b, n, o))


# speedup vs baseline: 8.2445x; 1.0191x over previous
import functools

import jax
import jax.numpy as jnp
from jax import lax
from jax.experimental import pallas as pl
from jax.experimental.pallas import tpu as pltpu
from jax.experimental.pallas import tpu_sc as plsc

_RADIUS = 0.2
_R2 = _RADIUS * _RADIUS
_EPS = 1e-5


def _select_body(xyz8_ref, xyzt8_ref, out_ref, *, n_total, bn_total, cb):
    b = pl.program_id(0)
    nb = pl.program_id(1)
    ctr = xyz8_ref[0]
    xt = xyzt8_ref[0]
    colid = lax.broadcasted_iota(jnp.int32, (cb, n_total), 1)
    rowid = lax.broadcasted_iota(jnp.int32, (cb, n_total), 0) + nb * cb
    d0 = xt[0:1, :] - ctr[:, 0:1]
    d1 = xt[1:2, :] - ctr[:, 1:2]
    d2 = xt[2:3, :] - ctr[:, 2:3]
    dist = (d0 * d0 + d1 * d1) + d2 * d2
    r2 = jnp.float32(_R2)
    big = jnp.float32(1e10)
    base = (dist > 1e-10) & (dist < r2)
    eye = colid == rowid
    sub = ((jnp.where(d0 >= 0.0, jnp.int32(4), jnp.int32(0))
            + jnp.where(d1 >= 0.0, jnp.int32(2), jnp.int32(0)))
           + jnp.where(d2 >= 0.0, jnp.int32(1), jnp.int32(0)))
    val_base = jnp.where(base, dist, big)
    val_base = jnp.where(eye, r2, val_base)
    for i in range(8):
        sel = (sub == i) | eye
        val = jnp.where(sel, val_base, big)
        mn = jnp.min(val, axis=1, keepdims=True)
        cand = jnp.where(val == mn, colid, jnp.int32(n_total))
        pj = jnp.min(cand, axis=1)
        out_ref[(i >> 1) + 4 * (i & 1), :] = (
            pj + (b * n_total + (i & 1) * bn_total))


def _select_call(xyz8, xyzt8, cb=512):
    b, n, _ = xyz8.shape
    nb = n // cb
    return pl.pallas_call(
        functools.partial(_select_body, n_total=n, bn_total=b * n, cb=cb),
        grid=(b, nb),
        in_specs=[
            pl.BlockSpec((1, cb, 8), lambda bi, ni: (bi, ni, 0)),
            pl.BlockSpec((1, 8, n), lambda bi, ni: (bi, 0, 0)),
        ],
        out_specs=pl.BlockSpec((8, cb), lambda bi, ni: (0, bi * nb + ni)),
        out_shape=jax.ShapeDtypeStruct((8, b * n), jnp.int32),
    )(xyz8, xyzt8)


def _pack_bf16(y):
    h = y.shape[-1] // 2
    yb = y.astype(jnp.bfloat16)
    lo = lax.bitcast_convert_type(yb[:, :h], jnp.uint16).astype(jnp.uint32)
    hi = lax.bitcast_convert_type(yb[:, h:], jnp.uint16).astype(jnp.uint32)
    return lo | (hi << 16)


def _unpack_bf16(g):
    lo = lax.bitcast_convert_type(g.astype(jnp.uint16), jnp.bfloat16)
    hi = lax.bitcast_convert_type((g >> 16).astype(jnp.uint16), jnp.bfloat16)
    return jnp.concatenate([lo, hi], axis=-1).astype(jnp.float32)


def _table_body(feat_ref, xyz8_ref, wf_ref, wx_ref, out_ref):
    y = (jnp.dot(feat_ref[...], wf_ref[0],
                 preferred_element_type=jnp.float32)
         + jnp.dot(xyz8_ref[...], wx_ref[0],
                   preferred_element_type=jnp.float32))
    out_ref[0] = _pack_bf16(y) if out_ref.dtype == jnp.uint32 else y


def _table_affine_body(y_ref, s_ref, t_ref, xyz8_ref, wf_ref, wx_ref, out_ref):
    z = jnp.maximum(y_ref[...] * s_ref[...] + t_ref[...], 0.0)
    y = (jnp.dot(z, wf_ref[0], preferred_element_type=jnp.float32)
         + jnp.dot(xyz8_ref[...], wx_ref[0],
                   preferred_element_type=jnp.float32))
    out_ref[0] = _pack_bf16(y) if out_ref.dtype == jnp.uint32 else y


def _table_call(feat, xyz8f, wf, wx, rb=512, pack=True):
    m, c = feat.shape
    o = wf.shape[-1]
    oo = o // 2 if pack else o
    odt = jnp.uint32 if pack else jnp.float32
    return pl.pallas_call(
        _table_body,
        grid=(2, m // rb),
        in_specs=[
            pl.BlockSpec((rb, c), lambda p, i: (i, 0)),
            pl.BlockSpec((rb, 8), lambda p, i: (i, 0)),
            pl.BlockSpec((1, c, o), lambda p, i: (p, 0, 0)),
            pl.BlockSpec((1, 8, o), lambda p, i: (p, 0, 0)),
        ],
        out_specs=pl.BlockSpec((1, rb, oo), lambda p, i: (p, i, 0)),
        out_shape=jax.ShapeDtypeStruct((2, m, oo), odt),
    )(feat, xyz8f, wf, wx)


def _table_affine_call(y, s, t, xyz8f, wf, wx, rb=512, pack=True):
    m, c = y.shape
    o = wf.shape[-1]
    oo = o // 2 if pack else o
    odt = jnp.uint32 if pack else jnp.float32
    return pl.pallas_call(
        _table_affine_body,
        grid=(2, m // rb),
        in_specs=[
            pl.BlockSpec((rb, c), lambda p, i: (i, 0)),
            pl.BlockSpec((1, c), lambda p, i: (0, 0)),
            pl.BlockSpec((1, c), lambda p, i: (0, 0)),
            pl.BlockSpec((rb, 8), lambda p, i: (i, 0)),
            pl.BlockSpec((1, c, o), lambda p, i: (p, 0, 0)),
            pl.BlockSpec((1, 8, o), lambda p, i: (p, 0, 0)),
        ],
        out_specs=pl.BlockSpec((1, rb, oo), lambda p, i: (p, i, 0)),
        out_shape=jax.ShapeDtypeStruct((2, m, oo), odt),
    )(y, s, t, xyz8f, wf, wx)


def _sc_gather_call(table, idx_flat):
    v, d = table.shape
    dt = table.dtype
    (r,) = idx_flat.shape
    nc, ns = 2, 16
    nw = nc * ns
    per_w = r // nw
    ch = 128
    nchunk = per_w // ch
    mesh = plsc.VectorSubcoreMesh(core_axis_name="c", subcore_axis_name="s")

    nbuf = 4

    @functools.partial(
        pl.kernel,
        out_type=jax.ShapeDtypeStruct((r, d), dt),
        mesh=mesh,
        scratch_types=(
            [pltpu.VMEM((per_w,), jnp.int32)]
            + [pltpu.VMEM((ch, d), dt) for _ in range(nbuf)]
            + [pltpu.SemaphoreType.DMA for _ in range(2 * nbuf)]
        ),
    )
    def gk(table_hbm, idx_hbm, out_hbm, idx_v, *bufs):
        row_v = bufs[:nbuf]
        gsem = bufs[nbuf:2 * nbuf]
        wsem = bufs[2 * nbuf:]
        wid = lax.axis_index("s") * nc + lax.axis_index("c")
        base = wid * per_w
        pltpu.sync_copy(idx_hbm.at[pl.ds(base, per_w)], idx_v)
        gh = [None] * nchunk
        wh = [None] * nchunk
        for i in range(min(nbuf, nchunk)):
            gh[i] = pltpu.async_copy(
                table_hbm.at[idx_v.at[pl.ds(i * ch, ch)]], row_v[i], gsem[i])
        for i in range(nchunk):
            cur = i % nbuf
            gh[i].wait()
            wh[i] = pltpu.async_copy(
                row_v[cur], out_hbm.at[pl.ds(base + i * ch, ch)], wsem[cur])
            if i + nbuf < nchunk:
                wh[i].wait()
                gh[i + nbuf] = pltpu.async_copy(
                    table_hbm.at[idx_v.at[pl.ds((i + nbuf) * ch, ch)]],
                    row_v[cur], gsem[cur])
        for i in range(max(0, nchunk - nbuf), nchunk):
            wh[i].wait()

    return gk(table, idx_flat)


def _gsum(ga_ref, gb_ref, xt_ref, wxn_ref, bias_ref):
    cc = lax.dot_general(xt_ref[...], wxn_ref[...],
                         (((0,), (0,)), ((), ())),
                         preferred_element_type=jnp.float32)
    ga = ga_ref[0, 0]
    gb = gb_ref[0, 0]
    if ga.dtype == jnp.uint32:
        ga = _unpack_bf16(ga)
        gb = _unpack_bf16(gb)
    return ga + gb + cc + bias_ref[...]


def _finalize_bn(acc, gam_ref, bet_ref, s_ref, t_ref, count):
    inv = jnp.float32(1.0 / count)
    mean = acc[0:1, :] * inv
    var = acc[1:2, :] * inv - mean * mean
    sc = gam_ref[...] / jnp.sqrt(var + _EPS)
    s_ref[...] = sc
    t_ref[...] = bet_ref[...] - mean * sc


def _gstats_body(ga_ref, gb_ref, xt_ref, wxn_ref, bias_ref, gam_ref, bet_ref,
                 s_ref, t_ref, acc, *, kdim, nsteps, count):
    k = pl.program_id(0)
    i = pl.program_id(1)

    @pl.when((k == 0) & (i == 0))
    def _init():
        acc[...] = jnp.zeros_like(acc)

    y = _gsum(ga_ref, gb_ref, xt_ref, wxn_ref, bias_ref)
    acc[0:1, :] = acc[0:1, :] + jnp.sum(y, axis=0, keepdims=True)
    acc[1:2, :] = acc[1:2, :] + jnp.sum(y * y, axis=0, keepdims=True)

    @pl.when((k == kdim - 1) & (i == nsteps - 1))
    def _fin():
        _finalize_bn(acc, gam_ref, bet_ref, s_ref, t_ref, count)


def _gstats_call(g4, xt, wxn, bias, gam, bet, rb=1024):
    _, kdim, m, gw = g4.shape
    d = bias.shape[-1]
    nsteps = m // rb
    count = kdim * m
    return pl.pallas_call(
        functools.partial(_gstats_body, kdim=kdim, nsteps=nsteps,
                          count=count),
        grid=(kdim, nsteps),
        in_specs=[
            pl.BlockSpec((1, 1, rb, gw), lambda k, i: (0, k, i, 0)),
            pl.BlockSpec((1, 1, rb, gw), lambda k, i: (1, k, i, 0)),
            pl.BlockSpec((8, rb), lambda k, i: (0, i)),
            pl.BlockSpec((8, d), lambda k, i: (0, 0)),
            pl.BlockSpec((1, d), lambda k, i: (0, 0)),
            pl.BlockSpec((1, d), lambda k, i: (0, 0)),
            pl.BlockSpec((1, d), lambda k, i: (0, 0)),
        ],
        out_specs=[
            pl.BlockSpec((1, d), lambda k, i: (0, 0)),
            pl.BlockSpec((1, d), lambda k, i: (0, 0)),
        ],
        out_shape=[
            jax.ShapeDtypeStruct((1, d), jnp.float32),
            jax.ShapeDtypeStruct((1, d), jnp.float32),
        ],
        scratch_shapes=[pltpu.VMEM((2, d), jnp.float32)],
    )(g4, g4, xt, wxn, bias, gam, bet)


def _dualmm_g_body(ga0_ref, gb0_ref, ga1_ref, gb1_ref, xt_ref, wxn_ref,
                   b1_ref, sp_ref, tp_ref, wa_ref, wb_ref, b2_ref,
                   gam_ref, bet_ref, o_ref, s_ref, t_ref, acc,
                   *, jdim, nsteps, count):
    j = pl.program_id(0)
    i = pl.program_id(1)

    @pl.when((j == 0) & (i == 0))
    def _init():
        acc[...] = jnp.zeros_like(acc)

    sp = sp_ref[...]
    tp = tp_ref[...]
    ya = _gsum(ga0_ref, gb0_ref, xt_ref, wxn_ref, b1_ref)
    yb = _gsum(ga1_ref, gb1_ref, xt_ref, wxn_ref, b1_ref)
    za = jnp.maximum(ya * sp + tp, 0.0)
    zb = jnp.maximum(yb * sp + tp, 0.0)
    y = (jnp.dot(za, wa_ref[0], preferred_element_type=jnp.float32)
         + jnp.dot(zb, wb_ref[0], preferred_element_type=jnp.float32)
         + b2_ref[...])
    o_ref[0] = y
    acc[0:1, :] = acc[0:1, :] + jnp.sum(y, axis=0, keepdims=True)
    acc[1:2, :] = acc[1:2, :] + jnp.sum(y * y, axis=0, keepdims=True)

    @pl.when((j == jdim - 1) & (i == nsteps - 1))
    def _fin():
        _finalize_bn(acc, gam_ref, bet_ref, s_ref, t_ref, count)


def _dualmm_g_call(g4, xt, wxn, b1, sp, tp, w2, b2, gam, bet, rb=1024):
    _, _, m, gw = g4.shape
    d = b2.shape[-1]
    jdim = 2
    nsteps = m // rb
    count = jdim * m
    return pl.pallas_call(
        functools.partial(_dualmm_g_body, jdim=jdim, nsteps=nsteps,
                          count=count),
        grid=(jdim, nsteps),
        in_specs=[
            pl.BlockSpec((1, 1, rb, gw), lambda j, i: (0, 2 * j, i, 0)),
            pl.BlockSpec((1, 1, rb, gw), lambda j, i: (1, 2 * j, i, 0)),
            pl.BlockSpec((1, 1, rb, gw), lambda j, i: (0, 2 * j + 1, i, 0)),
            pl.BlockSpec((1, 1, rb, gw), lambda j, i: (1, 2 * j + 1, i, 0)),
            pl.BlockSpec((8, rb), lambda j, i: (0, i)),
            pl.BlockSpec((8, d), lambda j, i: (0, 0)),
            pl.BlockSpec((1, d), lambda j, i: (0, 0)),
            pl.BlockSpec((1, d), lambda j, i: (0, 0)),
            pl.BlockSpec((1, d), lambda j, i: (0, 0)),
            pl.BlockSpec((1, d, d), lambda j, i: (0, 0, 0)),
            pl.BlockSpec((1, d, d), lambda j, i: (1, 0, 0)),
            pl.BlockSpec((1, d), lambda j, i: (0, 0)),
            pl.BlockSpec((1, d), lambda j, i: (0, 0)),
            pl.BlockSpec((1, d), lambda j, i: (0, 0)),
        ],
        out_specs=[
            pl.BlockSpec((1, rb, d), lambda j, i: (j, i, 0)),
            pl.BlockSpec((1, d), lambda j, i: (0, 0)),
            pl.BlockSpec((1, d), lambda j, i: (0, 0)),
        ],
        out_shape=[
            jax.ShapeDtypeStruct((jdim, m, d), jnp.float32),
            jax.ShapeDtypeStruct((1, d), jnp.float32),
            jax.ShapeDtypeStruct((1, d), jnp.float32),
        ],
        scratch_shapes=[pltpu.VMEM((2, d), jnp.float32)],
    )(g4, g4, g4, g4, xt, wxn, b1, sp, tp, w2, w2, b2, gam, bet)


def _dualmm_body(ya_ref, yb_ref, sp_ref, tp_ref, wa_ref, wb_ref, b_ref,
                 gam_ref, bet_ref, o_ref, s_ref, t_ref, acc,
                 *, jdim, nsteps, count):
    j = pl.program_id(0)
    i = pl.program_id(1)

    @pl.when((j == 0) & (i == 0))
    def _init():
        acc[...] = jnp.zeros_like(acc)

    sp = sp_ref[...]
    tp = tp_ref[...]
    za = jnp.maximum(ya_ref[0] * sp + tp, 0.0)
    zb = jnp.maximum(yb_ref[0] * sp + tp, 0.0)
    y = (jnp.dot(za, wa_ref[0], preferred_element_type=jnp.float32)
         + jnp.dot(zb, wb_ref[0], preferred_element_type=jnp.float32)
         + b_ref[...])
    o_ref[0] = y
    acc[0:1, :] = acc[0:1, :] + jnp.sum(y, axis=0, keepdims=True)
    acc[1:2, :] = acc[1:2, :] + jnp.sum(y * y, axis=0, keepdims=True)

    @pl.when((j == jdim - 1) & (i == nsteps - 1))
    def _fin():
        inv = jnp.float32(1.0 / count)
        mean = acc[0:1, :] * inv
        var = acc[1:2, :] * inv - mean * mean
        sc = gam_ref[...] / jnp.sqrt(var + _EPS)
        s_ref[...] = sc
        t_ref[...] = bet_ref[...] - mean * sc


def _dualmm_call(yk, sp, tp, w2, bias, gam, bet, rb=1024):
    kin, m, d = yk.shape
    jdim = kin // 2
    nsteps = m // rb
    count = jdim * m
    return pl.pallas_call(
        functools.partial(_dualmm_body, jdim=jdim, nsteps=nsteps,
                          count=count),
        grid=(jdim, nsteps),
        in_specs=[
            pl.BlockSpec((1, rb, d), lambda j, i: (2 * j, i, 0)),
            pl.BlockSpec((1, rb, d), lambda j, i: (2 * j + 1, i, 0)),
            pl.BlockSpec((1, d), lambda j, i: (0, 0)),
            pl.BlockSpec((1, d), lambda j, i: (0, 0)),
            pl.BlockSpec((1, d, d), lambda j, i: (0, 0, 0)),
            pl.BlockSpec((1, d, d), lambda j, i: (1, 0, 0)),
            pl.BlockSpec((1, d), lambda j, i: (0, 0)),
            pl.BlockSpec((1, d), lambda j, i: (0, 0)),
            pl.BlockSpec((1, d), lambda j, i: (0, 0)),
        ],
        out_specs=[
            pl.BlockSpec((1, rb, d), lambda j, i: (j, i, 0)),
            pl.BlockSpec((1, d), lambda j, i: (0, 0)),
            pl.BlockSpec((1, d), lambda j, i: (0, 0)),
        ],
        out_shape=[
            jax.ShapeDtypeStruct((jdim, m, d), jnp.float32),
            jax.ShapeDtypeStruct((1, d), jnp.float32),
            jax.ShapeDtypeStruct((1, d), jnp.float32),
        ],
        scratch_shapes=[pltpu.VMEM((2, d), jnp.float32)],
    )(yk, yk, sp, tp, w2, w2, bias, gam, bet)


def _final_body(y_ref, s_ref, t_ref, p_ref, o_ref):
    o_ref[...] = jnp.maximum(
        y_ref[...] * s_ref[...] + t_ref[...] + p_ref[...], 0.0)


def _final_call(y, s, t, pts, rb=1024):
    m, d = y.shape
    return pl.pallas_call(
        _final_body,
        grid=(m // rb,),
        in_specs=[
            pl.BlockSpec((rb, d), lambda i: (i, 0)),
            pl.BlockSpec((1, d), lambda i: (0, 0)),
            pl.BlockSpec((1, d), lambda i: (0, 0)),
            pl.BlockSpec((rb, d), lambda i: (i, 0)),
        ],
        out_specs=pl.BlockSpec((rb, d), lambda i: (i, 0)),
        out_shape=jax.ShapeDtypeStruct((m, d), jnp.float32),
    )(y, s, t, pts)


def _tap_weights(w):
    wx = jnp.stack([w[:, :3, 0, 0].T, w[:, :3, 0, 1].T])
    wx8 = jnp.pad(wx, ((0, 0), (0, 5), (0, 0)))
    wf = jnp.stack([w[:, 3:, 0, 0].T, w[:, 3:, 0, 1].T])
    return wx8, wf


def _pair_weight(w):
    return jnp.stack([w[:, :, 0, 0].T, w[:, :, 0, 1].T])


def kernel(xyz, points, conv1_params, conv2_params):
    b, n, _ = xyz.shape
    c = points.shape[-1]
    m = b * n

    xyz8 = jnp.pad(xyz, ((0, 0), (0, 0), (0, 5)))
    xyzt8 = jnp.transpose(xyz8, (0, 2, 1))
    idx_flat = _select_call(xyz8, xyzt8).reshape(-1)

    xyz8f = xyz8.reshape(m, 8)
    xt = jnp.transpose(xyz8, (2, 0, 1)).reshape(8, m)
    pts = points.reshape(m, c)

    (w1, b1, g1, be1), (w2, b2, g2, be2), (w3, b3, g3, be3) = conv1_params
    (v1, c1, h1, ce1), (v2, c2, h2, ce2), (v3, c3, h3, ce3) = conv2_params

    wx8, wf = _tap_weights(w1)
    tt = _table_call(pts, xyz8f, wf, wx8)
    gw = tt.shape[-1]
    o = b1.shape[0]
    gg = _sc_gather_call(tt.reshape(2 * m, gw), idx_flat)
    wxn = -(wx8[0] + wx8[1])
    gg4 = gg.reshape(2, 4, m, gw)
    s1, t1 = _gstats_call(gg4, xt, wxn, b1[None], g1[None], be1[None])
    y2, s2, t2 = _dualmm_g_call(gg4, xt, wxn, b1[None], s1, t1,
                                _pair_weight(w2), b2[None],
                                g2[None], be2[None])
    y3, s3, t3 = _dualmm_call(y2, s2, t2, _pair_weight(w3),
                              b3[None], g3[None], be3[None])

    vx8, vf = _tap_weights(v1)
    uu = _table_affine_call(y3.reshape(m, o), s3, t3, xyz8f, vf, vx8)
    hh = _sc_gather_call(uu.reshape(2 * m, gw), idx_flat)
    vxn = -(vx8[0] + vx8[1])
    hh4 = hh.reshape(2, 4, m, gw)
    s4, t4 = _gstats_call(hh4, xt, vxn, c1[None], h1[None], ce1[None])
    y5, s5, t5 = _dualmm_g_call(hh4, xt, vxn, c1[None], s4, t4,
                                _pair_weight(v2), c2[None],
                                h2[None], ce2[None])
    y6, s6, t6 = _dualmm_call(y5, s5, t5, _pair_weight(v3),
                              c3[None], h3[None], ce3[None])

    out = _final_call(y6.reshape(m, o), s6, t6, pts)
    return (xyz, out.reshape(b, n, o))


# speedup vs baseline: 8.8879x; 1.0780x over previous
import functools

import jax
import jax.numpy as jnp
from jax import lax
from jax.experimental import pallas as pl
from jax.experimental.pallas import tpu as pltpu
from jax.experimental.pallas import tpu_sc as plsc

_RADIUS = 0.2
_R2 = _RADIUS * _RADIUS
_EPS = 1e-5


def _select_body(xyz8_ref, xyzt8_ref, out_ref, *, n_total, bn_total, cb):
    b = pl.program_id(0)
    nb = pl.program_id(1)
    ctr = xyz8_ref[0]
    xt = xyzt8_ref[0]
    colid = lax.broadcasted_iota(jnp.int32, (cb, n_total), 1)
    rowid = lax.broadcasted_iota(jnp.int32, (cb, n_total), 0) + nb * cb
    d0 = xt[0:1, :] - ctr[:, 0:1]
    d1 = xt[1:2, :] - ctr[:, 1:2]
    d2 = xt[2:3, :] - ctr[:, 2:3]
    dist = (d0 * d0 + d1 * d1) + d2 * d2
    r2 = jnp.float32(_R2)
    big = jnp.float32(1e10)
    base = (dist > 1e-10) & (dist < r2)
    eye = colid == rowid
    sub = ((jnp.where(d0 >= 0.0, jnp.int32(4), jnp.int32(0))
            + jnp.where(d1 >= 0.0, jnp.int32(2), jnp.int32(0)))
           + jnp.where(d2 >= 0.0, jnp.int32(1), jnp.int32(0)))
    val_base = jnp.where(base, dist, big)
    val_base = jnp.where(eye, r2, val_base)
    for i in range(8):
        sel = (sub == i) | eye
        val = jnp.where(sel, val_base, big)
        mn = jnp.min(val, axis=1, keepdims=True)
        cand = jnp.where(val == mn, colid, jnp.int32(n_total))
        pj = jnp.min(cand, axis=1)
        out_ref[(i >> 1) + 4 * (i & 1), :] = (
            pj + (b * n_total + (i & 1) * bn_total))


def _select_call(xyz8, xyzt8, cb=512):
    b, n, _ = xyz8.shape
    nb = n // cb
    return pl.pallas_call(
        functools.partial(_select_body, n_total=n, bn_total=b * n, cb=cb),
        grid=(b, nb),
        in_specs=[
            pl.BlockSpec((1, cb, 8), lambda bi, ni: (bi, ni, 0)),
            pl.BlockSpec((1, 8, n), lambda bi, ni: (bi, 0, 0)),
        ],
        out_specs=pl.BlockSpec((8, cb), lambda bi, ni: (0, bi * nb + ni)),
        out_shape=jax.ShapeDtypeStruct((8, b * n), jnp.int32),
    )(xyz8, xyzt8)


def _pack_bf16(y):
    h = y.shape[-1] // 2
    yb = y.astype(jnp.bfloat16)
    lo = lax.bitcast_convert_type(yb[:, :h], jnp.uint16).astype(jnp.uint32)
    hi = lax.bitcast_convert_type(yb[:, h:], jnp.uint16).astype(jnp.uint32)
    return lo | (hi << 16)


def _unpack_bf16(g):
    lo = lax.bitcast_convert_type(g.astype(jnp.uint16), jnp.bfloat16)
    hi = lax.bitcast_convert_type((g >> 16).astype(jnp.uint16), jnp.bfloat16)
    return jnp.concatenate([lo, hi], axis=-1).astype(jnp.float32)


def _table_body(feat_ref, xyz8_ref, wf_ref, wx_ref, out_ref):
    y = (jnp.dot(feat_ref[...], wf_ref[0],
                 preferred_element_type=jnp.float32)
         + jnp.dot(xyz8_ref[...], wx_ref[0],
                   preferred_element_type=jnp.float32))
    out_ref[0] = _pack_bf16(y) if out_ref.dtype == jnp.uint32 else y


def _table_affine_body(y_ref, s_ref, t_ref, xyz8_ref, wf_ref, wx_ref, out_ref):
    z = jnp.maximum(y_ref[...] * s_ref[...] + t_ref[...], 0.0)
    y = (jnp.dot(z, wf_ref[0], preferred_element_type=jnp.float32)
         + jnp.dot(xyz8_ref[...], wx_ref[0],
                   preferred_element_type=jnp.float32))
    out_ref[0] = _pack_bf16(y) if out_ref.dtype == jnp.uint32 else y


def _table_call(feat, xyz8f, wf, wx, rb=512, pack=True):
    m, c = feat.shape
    o = wf.shape[-1]
    oo = o // 2 if pack else o
    odt = jnp.uint32 if pack else jnp.float32
    return pl.pallas_call(
        _table_body,
        grid=(2, m // rb),
        in_specs=[
            pl.BlockSpec((rb, c), lambda p, i: (i, 0)),
            pl.BlockSpec((rb, 8), lambda p, i: (i, 0)),
            pl.BlockSpec((1, c, o), lambda p, i: (p, 0, 0)),
            pl.BlockSpec((1, 8, o), lambda p, i: (p, 0, 0)),
        ],
        out_specs=pl.BlockSpec((1, rb, oo), lambda p, i: (p, i, 0)),
        out_shape=jax.ShapeDtypeStruct((2, m, oo), odt),
    )(feat, xyz8f, wf, wx)


def _table_affine_call(y, s, t, xyz8f, wf, wx, rb=512, pack=True):
    m, c = y.shape
    o = wf.shape[-1]
    oo = o // 2 if pack else o
    odt = jnp.uint32 if pack else jnp.float32
    return pl.pallas_call(
        _table_affine_body,
        grid=(2, m // rb),
        in_specs=[
            pl.BlockSpec((rb, c), lambda p, i: (i, 0)),
            pl.BlockSpec((1, c), lambda p, i: (0, 0)),
            pl.BlockSpec((1, c), lambda p, i: (0, 0)),
            pl.BlockSpec((rb, 8), lambda p, i: (i, 0)),
            pl.BlockSpec((1, c, o), lambda p, i: (p, 0, 0)),
            pl.BlockSpec((1, 8, o), lambda p, i: (p, 0, 0)),
        ],
        out_specs=pl.BlockSpec((1, rb, oo), lambda p, i: (p, i, 0)),
        out_shape=jax.ShapeDtypeStruct((2, m, oo), odt),
    )(y, s, t, xyz8f, wf, wx)


def _sc_gather_call(table, idx_flat):
    v, d = table.shape
    dt = table.dtype
    (r,) = idx_flat.shape
    nc, ns = 2, 16
    nw = nc * ns
    per_w = r // nw
    ch = 128
    nchunk = per_w // ch
    mesh = plsc.VectorSubcoreMesh(core_axis_name="c", subcore_axis_name="s")

    nbuf = 4

    @functools.partial(
        pl.kernel,
        out_type=jax.ShapeDtypeStruct((r, d), dt),
        mesh=mesh,
        scratch_types=(
            [pltpu.VMEM((per_w,), jnp.int32)]
            + [pltpu.VMEM((ch, d), dt) for _ in range(nbuf)]
            + [pltpu.SemaphoreType.DMA for _ in range(2 * nbuf)]
        ),
    )
    def gk(table_hbm, idx_hbm, out_hbm, idx_v, *bufs):
        row_v = bufs[:nbuf]
        gsem = bufs[nbuf:2 * nbuf]
        wsem = bufs[2 * nbuf:]
        wid = lax.axis_index("s") * nc + lax.axis_index("c")
        base = wid * per_w
        pltpu.sync_copy(idx_hbm.at[pl.ds(base, per_w)], idx_v)
        gh = [None] * nchunk
        wh = [None] * nchunk
        for i in range(min(nbuf, nchunk)):
            gh[i] = pltpu.async_copy(
                table_hbm.at[idx_v.at[pl.ds(i * ch, ch)]], row_v[i], gsem[i])
        for i in range(nchunk):
            cur = i % nbuf
            gh[i].wait()
            wh[i] = pltpu.async_copy(
                row_v[cur], out_hbm.at[pl.ds(base + i * ch, ch)], wsem[cur])
            if i + nbuf < nchunk:
                wh[i].wait()
                gh[i + nbuf] = pltpu.async_copy(
                    table_hbm.at[idx_v.at[pl.ds((i + nbuf) * ch, ch)]],
                    row_v[cur], gsem[cur])
        for i in range(max(0, nchunk - nbuf), nchunk):
            wh[i].wait()

    return gk(table, idx_flat)


def _gsum(ga_ref, gb_ref, xt_ref, wxn_ref, bias_ref):
    cc = lax.dot_general(xt_ref[...], wxn_ref[...],
                         (((0,), (0,)), ((), ())),
                         preferred_element_type=jnp.float32)
    ga = ga_ref[0, 0]
    gb = gb_ref[0, 0]
    if ga.dtype == jnp.uint32:
        ga = _unpack_bf16(ga)
        gb = _unpack_bf16(gb)
    return ga + gb + cc + bias_ref[...]


def _finalize_bn(acc, gam_ref, bet_ref, s_ref, t_ref, count):
    inv = jnp.float32(1.0 / count)
    mean = acc[0:1, :] * inv
    var = acc[1:2, :] * inv - mean * mean
    sc = gam_ref[...] / jnp.sqrt(var + _EPS)
    s_ref[...] = sc
    t_ref[...] = bet_ref[...] - mean * sc


def _gstats_body(ga_ref, gb_ref, xt_ref, wxn_ref, bias_ref, gam_ref, bet_ref,
                 s_ref, t_ref, acc, *, kdim, nsteps, count):
    k = pl.program_id(0)
    i = pl.program_id(1)

    @pl.when((k == 0) & (i == 0))
    def _init():
        acc[...] = jnp.zeros_like(acc)

    y = _gsum(ga_ref, gb_ref, xt_ref, wxn_ref, bias_ref)
    acc[0:1, :] = acc[0:1, :] + jnp.sum(y, axis=0, keepdims=True)
    acc[1:2, :] = acc[1:2, :] + jnp.sum(y * y, axis=0, keepdims=True)

    @pl.when((k == kdim - 1) & (i == nsteps - 1))
    def _fin():
        _finalize_bn(acc, gam_ref, bet_ref, s_ref, t_ref, count)


def _gstats_call(g4, xt, wxn, bias, gam, bet, rb=2048):
    _, kdim, m, gw = g4.shape
    d = bias.shape[-1]
    nsteps = m // rb
    count = kdim * m
    return pl.pallas_call(
        functools.partial(_gstats_body, kdim=kdim, nsteps=nsteps,
                          count=count),
        grid=(kdim, nsteps),
        in_specs=[
            pl.BlockSpec((1, 1, rb, gw), lambda k, i: (0, k, i, 0)),
            pl.BlockSpec((1, 1, rb, gw), lambda k, i: (1, k, i, 0)),
            pl.BlockSpec((8, rb), lambda k, i: (0, i)),
            pl.BlockSpec((8, d), lambda k, i: (0, 0)),
            pl.BlockSpec((1, d), lambda k, i: (0, 0)),
            pl.BlockSpec((1, d), lambda k, i: (0, 0)),
            pl.BlockSpec((1, d), lambda k, i: (0, 0)),
        ],
        out_specs=[
            pl.BlockSpec((1, d), lambda k, i: (0, 0)),
            pl.BlockSpec((1, d), lambda k, i: (0, 0)),
        ],
        out_shape=[
            jax.ShapeDtypeStruct((1, d), jnp.float32),
            jax.ShapeDtypeStruct((1, d), jnp.float32),
        ],
        scratch_shapes=[pltpu.VMEM((2, d), jnp.float32)],
    )(g4, g4, xt, wxn, bias, gam, bet)


def _dualmm_g_body(ga0_ref, gb0_ref, ga1_ref, gb1_ref, xt_ref, wxn_ref,
                   b1_ref, sp_ref, tp_ref, wa_ref, wb_ref, b2_ref,
                   gam_ref, bet_ref, o_ref, s_ref, t_ref, acc,
                   *, jdim, nsteps, count):
    j = pl.program_id(0)
    i = pl.program_id(1)

    @pl.when((j == 0) & (i == 0))
    def _init():
        acc[...] = jnp.zeros_like(acc)

    sp = sp_ref[...]
    tp = tp_ref[...]
    ya = _gsum(ga0_ref, gb0_ref, xt_ref, wxn_ref, b1_ref)
    yb = _gsum(ga1_ref, gb1_ref, xt_ref, wxn_ref, b1_ref)
    za = jnp.maximum(ya * sp + tp, 0.0)
    zb = jnp.maximum(yb * sp + tp, 0.0)
    y = (jnp.dot(za, wa_ref[0], preferred_element_type=jnp.float32)
         + jnp.dot(zb, wb_ref[0], preferred_element_type=jnp.float32)
         + b2_ref[...])
    o_ref[0] = y
    acc[0:1, :] = acc[0:1, :] + jnp.sum(y, axis=0, keepdims=True)
    acc[1:2, :] = acc[1:2, :] + jnp.sum(y * y, axis=0, keepdims=True)

    @pl.when((j == jdim - 1) & (i == nsteps - 1))
    def _fin():
        _finalize_bn(acc, gam_ref, bet_ref, s_ref, t_ref, count)


def _dualmm_g_call(g4, xt, wxn, b1, sp, tp, w2, b2, gam, bet, rb=2048):
    _, _, m, gw = g4.shape
    d = b2.shape[-1]
    jdim = 2
    nsteps = m // rb
    count = jdim * m
    return pl.pallas_call(
        functools.partial(_dualmm_g_body, jdim=jdim, nsteps=nsteps,
                          count=count),
        grid=(jdim, nsteps),
        in_specs=[
            pl.BlockSpec((1, 1, rb, gw), lambda j, i: (0, 2 * j, i, 0)),
            pl.BlockSpec((1, 1, rb, gw), lambda j, i: (1, 2 * j, i, 0)),
            pl.BlockSpec((1, 1, rb, gw), lambda j, i: (0, 2 * j + 1, i, 0)),
            pl.BlockSpec((1, 1, rb, gw), lambda j, i: (1, 2 * j + 1, i, 0)),
            pl.BlockSpec((8, rb), lambda j, i: (0, i)),
            pl.BlockSpec((8, d), lambda j, i: (0, 0)),
            pl.BlockSpec((1, d), lambda j, i: (0, 0)),
            pl.BlockSpec((1, d), lambda j, i: (0, 0)),
            pl.BlockSpec((1, d), lambda j, i: (0, 0)),
            pl.BlockSpec((1, d, d), lambda j, i: (0, 0, 0)),
            pl.BlockSpec((1, d, d), lambda j, i: (1, 0, 0)),
            pl.BlockSpec((1, d), lambda j, i: (0, 0)),
            pl.BlockSpec((1, d), lambda j, i: (0, 0)),
            pl.BlockSpec((1, d), lambda j, i: (0, 0)),
        ],
        out_specs=[
            pl.BlockSpec((1, rb, d), lambda j, i: (j, i, 0)),
            pl.BlockSpec((1, d), lambda j, i: (0, 0)),
            pl.BlockSpec((1, d), lambda j, i: (0, 0)),
        ],
        out_shape=[
            jax.ShapeDtypeStruct((jdim, m, d), jnp.float32),
            jax.ShapeDtypeStruct((1, d), jnp.float32),
            jax.ShapeDtypeStruct((1, d), jnp.float32),
        ],
        scratch_shapes=[pltpu.VMEM((2, d), jnp.float32)],
    )(g4, g4, g4, g4, xt, wxn, b1, sp, tp, w2, w2, b2, gam, bet)


def _dualmm_body(ya_ref, yb_ref, sp_ref, tp_ref, wa_ref, wb_ref, b_ref,
                 gam_ref, bet_ref, o_ref, s_ref, t_ref, acc,
                 *, jdim, nsteps, count):
    j = pl.program_id(0)
    i = pl.program_id(1)

    @pl.when((j == 0) & (i == 0))
    def _init():
        acc[...] = jnp.zeros_like(acc)

    sp = sp_ref[...]
    tp = tp_ref[...]
    za = jnp.maximum(ya_ref[0] * sp + tp, 0.0)
    zb = jnp.maximum(yb_ref[0] * sp + tp, 0.0)
    y = (jnp.dot(za, wa_ref[0], preferred_element_type=jnp.float32)
         + jnp.dot(zb, wb_ref[0], preferred_element_type=jnp.float32)
         + b_ref[...])
    o_ref[0] = y
    acc[0:1, :] = acc[0:1, :] + jnp.sum(y, axis=0, keepdims=True)
    acc[1:2, :] = acc[1:2, :] + jnp.sum(y * y, axis=0, keepdims=True)

    @pl.when((j == jdim - 1) & (i == nsteps - 1))
    def _fin():
        inv = jnp.float32(1.0 / count)
        mean = acc[0:1, :] * inv
        var = acc[1:2, :] * inv - mean * mean
        sc = gam_ref[...] / jnp.sqrt(var + _EPS)
        s_ref[...] = sc
        t_ref[...] = bet_ref[...] - mean * sc


def _dualmm_call(yk, sp, tp, w2, bias, gam, bet, rb=2048):
    kin, m, d = yk.shape
    jdim = kin // 2
    nsteps = m // rb
    count = jdim * m
    return pl.pallas_call(
        functools.partial(_dualmm_body, jdim=jdim, nsteps=nsteps,
                          count=count),
        grid=(jdim, nsteps),
        in_specs=[
            pl.BlockSpec((1, rb, d), lambda j, i: (2 * j, i, 0)),
            pl.BlockSpec((1, rb, d), lambda j, i: (2 * j + 1, i, 0)),
            pl.BlockSpec((1, d), lambda j, i: (0, 0)),
            pl.BlockSpec((1, d), lambda j, i: (0, 0)),
            pl.BlockSpec((1, d, d), lambda j, i: (0, 0, 0)),
            pl.BlockSpec((1, d, d), lambda j, i: (1, 0, 0)),
            pl.BlockSpec((1, d), lambda j, i: (0, 0)),
            pl.BlockSpec((1, d), lambda j, i: (0, 0)),
            pl.BlockSpec((1, d), lambda j, i: (0, 0)),
        ],
        out_specs=[
            pl.BlockSpec((1, rb, d), lambda j, i: (j, i, 0)),
            pl.BlockSpec((1, d), lambda j, i: (0, 0)),
            pl.BlockSpec((1, d), lambda j, i: (0, 0)),
        ],
        out_shape=[
            jax.ShapeDtypeStruct((jdim, m, d), jnp.float32),
            jax.ShapeDtypeStruct((1, d), jnp.float32),
            jax.ShapeDtypeStruct((1, d), jnp.float32),
        ],
        scratch_shapes=[pltpu.VMEM((2, d), jnp.float32)],
    )(yk, yk, sp, tp, w2, w2, bias, gam, bet)


def _final_body(y_ref, s_ref, t_ref, p_ref, o_ref):
    o_ref[...] = jnp.maximum(
        y_ref[...] * s_ref[...] + t_ref[...] + p_ref[...], 0.0)


def _final_call(y, s, t, pts, rb=2048):
    m, d = y.shape
    return pl.pallas_call(
        _final_body,
        grid=(m // rb,),
        in_specs=[
            pl.BlockSpec((rb, d), lambda i: (i, 0)),
            pl.BlockSpec((1, d), lambda i: (0, 0)),
            pl.BlockSpec((1, d), lambda i: (0, 0)),
            pl.BlockSpec((rb, d), lambda i: (i, 0)),
        ],
        out_specs=pl.BlockSpec((rb, d), lambda i: (i, 0)),
        out_shape=jax.ShapeDtypeStruct((m, d), jnp.float32),
    )(y, s, t, pts)


def _tap_weights(w):
    wx = jnp.stack([w[:, :3, 0, 0].T, w[:, :3, 0, 1].T])
    wx8 = jnp.pad(wx, ((0, 0), (0, 5), (0, 0)))
    wf = jnp.stack([w[:, 3:, 0, 0].T, w[:, 3:, 0, 1].T])
    return wx8, wf


def _pair_weight(w):
    return jnp.stack([w[:, :, 0, 0].T, w[:, :, 0, 1].T])


def kernel(xyz, points, conv1_params, conv2_params):
    b, n, _ = xyz.shape
    c = points.shape[-1]
    m = b * n

    xyz8 = jnp.pad(xyz, ((0, 0), (0, 0), (0, 5)))
    xyzt8 = jnp.transpose(xyz8, (0, 2, 1))
    idx_flat = _select_call(xyz8, xyzt8).reshape(-1)

    xyz8f = xyz8.reshape(m, 8)
    xt = jnp.transpose(xyz8, (2, 0, 1)).reshape(8, m)
    pts = points.reshape(m, c)

    (w1, b1, g1, be1), (w2, b2, g2, be2), (w3, b3, g3, be3) = conv1_params
    (v1, c1, h1, ce1), (v2, c2, h2, ce2), (v3, c3, h3, ce3) = conv2_params

    wx8, wf = _tap_weights(w1)
    tt = _table_call(pts, xyz8f, wf, wx8)
    gw = tt.shape[-1]
    o = b1.shape[0]
    gg = _sc_gather_call(tt.reshape(2 * m, gw), idx_flat)
    wxn = -(wx8[0] + wx8[1])
    gg4 = gg.reshape(2, 4, m, gw)
    s1, t1 = _gstats_call(gg4, xt, wxn, b1[None], g1[None], be1[None])
    y2, s2, t2 = _dualmm_g_call(gg4, xt, wxn, b1[None], s1, t1,
                                _pair_weight(w2), b2[None],
                                g2[None], be2[None])
    y3, s3, t3 = _dualmm_call(y2, s2, t2, _pair_weight(w3),
                              b3[None], g3[None], be3[None])

    vx8, vf = _tap_weights(v1)
    uu = _table_affine_call(y3.reshape(m, o), s3, t3, xyz8f, vf, vx8)
    hh = _sc_gather_call(uu.reshape(2 * m, gw), idx_flat)
    vxn = -(vx8[0] + vx8[1])
    hh4 = hh.reshape(2, 4, m, gw)
    s4, t4 = _gstats_call(hh4, xt, vxn, c1[None], h1[None], ce1[None])
    y5, s5, t5 = _dualmm_g_call(hh4, xt, vxn, c1[None], s4, t4,
                                _pair_weight(v2), c2[None],
                                h2[None], ce2[None])
    y6, s6, t6 = _dualmm_call(y5, s5, t5, _pair_weight(v3),
                              c3[None], h3[None], ce3[None])

    out = _final_call(y6.reshape(m, o), s6, t6, pts)
    return (xyz, out.reshape(b, n, o))
